# Initial kernel scaffold; baseline (speedup 1.0000x reference)
import jax
import jax.numpy as jnp
from jax.experimental import pallas as pl


def kernel(x, edge_index, W1, b1, g1, be1, W2, b2, g2, be2, W3, b3, in_w, in_b, out_w, out_b):
    raise NotImplementedError("write your pallas kernel here")


# speedup vs baseline: 15.2815x; 15.2815x over previous
import functools

import jax
import jax.numpy as jnp
from jax import lax
from jax.experimental import pallas as pl
from jax.experimental.pallas import tpu as pltpu
from jax.experimental.pallas import tpu_sc as plsc

NC = 2
NS = 16
CH = 128
KW = 2
WE = CH * KW


def _cdiv(a, b):
    return (a + b - 1) // b


def _make_deg(n_pad, e_pad):
    ept = e_pad // (NC * NS)
    n_win = ept // WE
    rpt = n_pad // NS
    mesh = plsc.VectorSubcoreMesh(core_axis_name="c", subcore_axis_name="s")

    @functools.partial(
        pl.kernel,
        out_type=(jax.ShapeDtypeStruct((n_pad,), jnp.float32),
                  jax.ShapeDtypeStruct((n_pad,), jnp.float32)),
        mesh=mesh,
        scratch_types=[
            pltpu.VMEM_SHARED((n_pad,), jnp.float32),
            pltpu.VMEM((KW, CH), jnp.int32),
            pltpu.VMEM((CH,), jnp.float32),
            pltpu.VMEM((rpt,), jnp.float32),
        ],
    )
    def deg(dstr, onesr, zerr, out0, out1, acc, dstb, onesb, stage):
        c = lax.axis_index("c")
        s = lax.axis_index("s")
        pltpu.sync_copy(onesr, onesb)
        pltpu.sync_copy(zerr, stage)
        pltpu.sync_copy(stage, acc.at[pl.ds(s * rpt, rpt)])
        plsc.subcore_barrier()

        rbase = (c * NS + s) * (ept // CH)

        def win(w, carry):
            roff = rbase + w * KW
            pltpu.sync_copy(dstr.at[pl.ds(roff, KW)], dstb)
            for j in range(KW):
                pltpu.sync_copy(onesb, acc.at[dstb.at[j]], add=True)
            return carry

        lax.fori_loop(0, n_win, win, 0)
        plsc.subcore_barrier()
        pltpu.sync_copy(acc.at[pl.ds(s * rpt, rpt)], stage)

        @pl.when(c == 0)
        def _():
            pltpu.sync_copy(stage, out0.at[pl.ds(s * rpt, rpt)])

        @pl.when(c == 1)
        def _():
            pltpu.sync_copy(stage, out1.at[pl.ds(s * rpt, rpt)])

    return deg


def _make_agg(n_pad, d, e_pad):
    ept = e_pad // (NC * NS)
    n_win = ept // WE
    rpt = n_pad // NS
    chunks = [(off, min(WE, rpt - off)) for off in range(0, rpt, WE)]
    mesh = plsc.VectorSubcoreMesh(core_axis_name="c", subcore_axis_name="s")

    @functools.partial(
        pl.kernel,
        out_type=(jax.ShapeDtypeStruct((n_pad, d), jnp.float32),
                  jax.ShapeDtypeStruct((n_pad, d), jnp.float32)),
        mesh=mesh,
        scratch_types=[
            pltpu.VMEM_SHARED((n_pad, d), jnp.float32),
            pltpu.VMEM((KW, CH), jnp.int32),
            pltpu.VMEM((KW, CH), jnp.int32),
            pltpu.VMEM((WE, d), jnp.float32),
            pltpu.SemaphoreType.DMA,
        ],
    )
    def agg(table, srcr, dstr, zerr, out0, out1, acc, srcb, dstb, rows, sem):
        c = lax.axis_index("c")
        s = lax.axis_index("s")
        pltpu.sync_copy(zerr, rows)
        for off, sz in chunks:
            pltpu.sync_copy(rows.at[pl.ds(0, sz)],
                            acc.at[pl.ds(s * rpt + off, sz)])
        plsc.subcore_barrier()

        rbase = (c * NS + s) * (ept // CH)

        def win(w, carry):
            roff = rbase + w * KW
            pltpu.sync_copy(srcr.at[pl.ds(roff, KW)], srcb)
            pltpu.sync_copy(dstr.at[pl.ds(roff, KW)], dstb)
            for j in range(KW):
                pltpu.async_copy(table.at[srcb.at[j]],
                                 rows.at[pl.ds(j * CH, CH)], sem).wait()
            for j in range(KW):
                pltpu.sync_copy(rows.at[pl.ds(j * CH, CH)],
                                acc.at[dstb.at[j]], add=True)
            return carry

        lax.fori_loop(0, n_win, win, 0)
        plsc.subcore_barrier()

        for off, sz in chunks:
            pltpu.sync_copy(acc.at[pl.ds(s * rpt + off, sz)],
                            rows.at[pl.ds(0, sz)])

            @pl.when(c == 0)
            def _():
                pltpu.sync_copy(rows.at[pl.ds(0, sz)],
                                out0.at[pl.ds(s * rpt + off, sz)])

            @pl.when(c == 1)
            def _():
                pltpu.sync_copy(rows.at[pl.ds(0, sz)],
                                out1.at[pl.ds(s * rpt + off, sz)])

    return agg


_BR = 1000


def _mm_kernel(x_ref, w_ref, o_ref):
    o_ref[...] = jnp.dot(x_ref[...], w_ref[...],
                         preferred_element_type=jnp.float32)


def _tc_matmul(x, w):
    n, din = x.shape
    dout = w.shape[1]
    grid = (n // _BR,)
    return pl.pallas_call(
        _mm_kernel,
        grid=grid,
        in_specs=[pl.BlockSpec((_BR, din), lambda i: (i, 0)),
                  pl.BlockSpec((din, dout), lambda i: (0, 0))],
        out_specs=pl.BlockSpec((_BR, dout), lambda i: (i, 0)),
        out_shape=jax.ShapeDtypeStruct((n, dout), jnp.float32),
    )(x, w)


def _dinv_kernel(d0_ref, d1_ref, h_ref, dinv_ref, hp_ref):
    deg = d0_ref[...] + d1_ref[...] + 1.0
    dinv = 1.0 / jnp.sqrt(deg)
    dinv_ref[...] = dinv
    hp_ref[...] = h_ref[...] * dinv


def _tc_dinv_scale(deg0, deg1, h):
    n, din = h.shape
    grid = (n // _BR,)
    return pl.pallas_call(
        _dinv_kernel,
        grid=grid,
        in_specs=[pl.BlockSpec((_BR, 1), lambda i: (i, 0)),
                  pl.BlockSpec((_BR, 1), lambda i: (i, 0)),
                  pl.BlockSpec((_BR, din), lambda i: (i, 0))],
        out_specs=[pl.BlockSpec((_BR, 1), lambda i: (i, 0)),
                   pl.BlockSpec((_BR, din), lambda i: (i, 0))],
        out_shape=[jax.ShapeDtypeStruct((n, 1), jnp.float32),
                   jax.ShapeDtypeStruct((n, din), jnp.float32)],
    )(deg0, deg1, h)


_BN_SCALE = 1.0 / (1.0 + 1e-5) ** 0.5


def _mid_kernel(with_mm, *refs):
    if with_mm:
        p0_ref, p1_ref, hp_ref, dinv_ref, b_ref, g_ref, be_ref, w_ref, \
            o_ref = refs
    else:
        p0_ref, p1_ref, hp_ref, dinv_ref, b_ref, g_ref, be_ref, o_ref = refs
    dinv = dinv_ref[...]
    t = (p0_ref[...] + p1_ref[...] + hp_ref[...]) * dinv + b_ref[...]
    u = jnp.maximum(g_ref[...] * t * _BN_SCALE + be_ref[...], 0.0)
    if with_mm:
        u = jnp.dot(u, w_ref[...], preferred_element_type=jnp.float32)
    o_ref[...] = u * dinv


def _tc_mid(p0, p1, hp, dinv, b, g, be, w=None):
    n, din = hp.shape
    dout = w.shape[1] if w is not None else din
    grid = (n // _BR,)
    in_specs = [pl.BlockSpec((_BR, din), lambda i: (i, 0)),
                pl.BlockSpec((_BR, din), lambda i: (i, 0)),
                pl.BlockSpec((_BR, din), lambda i: (i, 0)),
                pl.BlockSpec((_BR, 1), lambda i: (i, 0)),
                pl.BlockSpec((1, din), lambda i: (0, 0)),
                pl.BlockSpec((1, din), lambda i: (0, 0)),
                pl.BlockSpec((1, din), lambda i: (0, 0))]
    args = [p0, p1, hp, dinv, b, g, be]
    if w is not None:
        in_specs.append(pl.BlockSpec((din, dout), lambda i: (0, 0)))
        args.append(w)
    return pl.pallas_call(
        functools.partial(_mid_kernel, w is not None),
        grid=grid,
        in_specs=in_specs,
        out_specs=pl.BlockSpec((_BR, dout), lambda i: (i, 0)),
        out_shape=jax.ShapeDtypeStruct((n, dout), jnp.float32),
    )(*args)


def _final_kernel(p0_ref, p1_ref, hp_ref, dinv_ref, w3_ref, b_ref, wv_ref,
                  bv_ref, ow_ref, ob_ref, o_ref):
    t = (p0_ref[...] + p1_ref[...] + hp_ref[...]) * dinv_ref[...]
    h = jnp.dot(t, w3_ref[...], preferred_element_type=jnp.float32) \
        + b_ref[...]
    v = jnp.dot(h, wv_ref[...], preferred_element_type=jnp.float32) \
        + bv_ref[...]
    o_ref[...] = h + jnp.dot(v, ow_ref[...],
                             preferred_element_type=jnp.float32) + ob_ref[...]


def _tc_final(p0, p1, hp, dinv, w3, b, wv_t, bv, ow_t, ob):
    n, din = hp.shape
    d = w3.shape[1]
    grid = (n // _BR,)
    return pl.pallas_call(
        _final_kernel,
        grid=grid,
        in_specs=[pl.BlockSpec((_BR, din), lambda i: (i, 0)),
                  pl.BlockSpec((_BR, din), lambda i: (i, 0)),
                  pl.BlockSpec((_BR, din), lambda i: (i, 0)),
                  pl.BlockSpec((_BR, 1), lambda i: (i, 0)),
                  pl.BlockSpec((din, d), lambda i: (0, 0)),
                  pl.BlockSpec((1, d), lambda i: (0, 0)),
                  pl.BlockSpec((d, d), lambda i: (0, 0)),
                  pl.BlockSpec((1, d), lambda i: (0, 0)),
                  pl.BlockSpec((d, d), lambda i: (0, 0)),
                  pl.BlockSpec((1, d), lambda i: (0, 0))],
        out_specs=pl.BlockSpec((_BR, d), lambda i: (i, 0)),
        out_shape=jax.ShapeDtypeStruct((n, d), jnp.float32),
    )(p0, p1, hp, dinv, w3, b, wv_t, bv, ow_t, ob)


def kernel(x, edge_index, W1, b1, g1, be1, W2, b2, g2, be2, W3, b3,
           in_w, in_b, out_w, out_b):
    n, d_in = x.shape
    e = edge_index.shape[1]
    d_out = W3.shape[1]

    tile_e = NC * NS * WE
    e_pad = _cdiv(e, tile_e) * tile_e
    n_pad = _cdiv(n, NS * 8) * NS * 8 + NS * 8
    rpt = n_pad // NS
    assert rpt % 8 == 0

    src = edge_index[0].astype(jnp.int32)
    dst = edge_index[1].astype(jnp.int32)
    pad_i = jnp.arange(e_pad - e, dtype=jnp.int32)
    src_p = jnp.concatenate([src, pad_i % n]).reshape(e_pad // CH, CH)
    dst_p = jnp.concatenate([dst, n + pad_i % (n_pad - n)]).reshape(
        e_pad // CH, CH)

    ones_r = jnp.ones((CH,), jnp.float32)
    zer1 = jnp.zeros((rpt,), jnp.float32)
    zer_w = jnp.zeros((WE, d_in), jnp.float32)

    deg0, deg1 = _make_deg(n_pad, e_pad)(dst_p, ones_r, zer1)
    deg0 = deg0[:n].reshape(n, 1)
    deg1 = deg1[:n].reshape(n, 1)

    h1 = _tc_matmul(x, W1)
    dinv, h1p = _tc_dinv_scale(deg0, deg1, h1)

    agg_w = _make_agg(n_pad, d_in, e_pad)

    p0, p1 = agg_w(h1p, src_p, dst_p, zer_w)
    h2p = _tc_mid(p0, p1, h1p, dinv, b1.reshape(1, -1), g1.reshape(1, -1),
                  be1.reshape(1, -1), W2)

    p0, p1 = agg_w(h2p, src_p, dst_p, zer_w)
    h3t = _tc_mid(p0, p1, h2p, dinv, b2.reshape(1, -1), g2.reshape(1, -1),
                  be2.reshape(1, -1))

    p0, p1 = agg_w(h3t, src_p, dst_p, zer_w)

    wv_t = in_w[2 * d_out:3 * d_out, :].T
    bv = in_b[2 * d_out:3 * d_out].reshape(1, -1)
    out = _tc_final(p0, p1, h3t, dinv, W3, b3.reshape(1, -1), wv_t, bv,
                    out_w.T, out_b.reshape(1, -1))
    return out


# speedup vs baseline: 24.4827x; 1.6021x over previous
import functools

import jax
import jax.numpy as jnp
from jax import lax
from jax.experimental import pallas as pl
from jax.experimental.pallas import tpu as pltpu
from jax.experimental.pallas import tpu_sc as plsc

NC = 2
NS = 16
CH = 128
KW = 2
WE = CH * KW
SBW = 8
NSB_STEP = 2 * SBW * CH


def _cdiv(a, b):
    return (a + b - 1) // b


def _make_deg(n_pad, e_pad):
    ept = e_pad // (NC * NS)
    n_win = ept // WE
    rpt = n_pad // NS
    mesh = plsc.VectorSubcoreMesh(core_axis_name="c", subcore_axis_name="s")

    @functools.partial(
        pl.kernel,
        out_type=(jax.ShapeDtypeStruct((n_pad,), jnp.float32),
                  jax.ShapeDtypeStruct((n_pad,), jnp.float32)),
        mesh=mesh,
        scratch_types=[
            pltpu.VMEM_SHARED((n_pad,), jnp.float32),
            pltpu.VMEM((KW, CH), jnp.int32),
            pltpu.VMEM((CH,), jnp.float32),
            pltpu.VMEM((rpt,), jnp.float32),
        ],
    )
    def deg(dstr, onesr, zerr, out0, out1, acc, dstb, onesb, stage):
        c = lax.axis_index("c")
        s = lax.axis_index("s")
        pltpu.sync_copy(onesr, onesb)
        pltpu.sync_copy(zerr, stage)
        pltpu.sync_copy(stage, acc.at[pl.ds(s * rpt, rpt)])
        plsc.subcore_barrier()

        rbase = (c * NS + s) * (ept // CH)

        def win(w, carry):
            roff = rbase + w * KW
            pltpu.sync_copy(dstr.at[pl.ds(roff, KW)], dstb)
            for j in range(KW):
                pltpu.sync_copy(onesb, acc.at[dstb.at[j]], add=True)
            return carry

        lax.fori_loop(0, n_win, win, 0)
        plsc.subcore_barrier()
        pltpu.sync_copy(acc.at[pl.ds(s * rpt, rpt)], stage)

        @pl.when(c == 0)
        def _():
            pltpu.sync_copy(stage, out0.at[pl.ds(s * rpt, rpt)])

        @pl.when(c == 1)
        def _():
            pltpu.sync_copy(stage, out1.at[pl.ds(s * rpt, rpt)])

    return deg


def _make_agg(n_pad, d, e_pad):
    ept = e_pad // (NC * NS)
    n_win = ept // CH
    nsb = n_win // SBW
    assert nsb % 2 == 0 and nsb * SBW == n_win
    rpt = n_pad // NS
    stg = 2 * CH
    chunks = [(off, min(stg, rpt - off)) for off in range(0, rpt, stg)]
    mesh = plsc.VectorSubcoreMesh(core_axis_name="c", subcore_axis_name="s")

    @functools.partial(
        pl.kernel,
        out_type=(jax.ShapeDtypeStruct((n_pad, d), jnp.float32),
                  jax.ShapeDtypeStruct((n_pad, d), jnp.float32)),
        mesh=mesh,
        scratch_types=[
            pltpu.VMEM_SHARED((n_pad, d), jnp.float32),
            pltpu.VMEM((2, SBW, CH), jnp.int32),
            pltpu.VMEM((2, SBW, CH), jnp.int32),
            pltpu.VMEM((stg, d), jnp.float32),
            pltpu.SemaphoreType.DMA,
            pltpu.SemaphoreType.DMA,
            pltpu.SemaphoreType.DMA,
        ],
    )
    def agg(table, srcr, dstr, zerr, out0, out1, acc, srcb, dstb, rows,
            isem, gsem, ssem):
        c = lax.axis_index("c")
        s = lax.axis_index("s")
        pltpu.sync_copy(zerr, rows)
        for off, sz in chunks:
            pltpu.sync_copy(rows.at[pl.ds(0, sz)],
                            acc.at[pl.ds(s * rpt + off, sz)])
        plsc.subcore_barrier()

        rbase = (c * NS + s) * n_win

        pltpu.sync_copy(srcr.at[pl.ds(rbase, SBW)], srcb.at[0])
        pltpu.sync_copy(dstr.at[pl.ds(rbase, SBW)], dstb.at[0])

        def outer(o, carry):
            for p in (0, 1):
                sb = 2 * o + p
                nxt = lax.rem(sb + 1, nsb)
                noff = rbase + nxt * SBW
                i0 = pltpu.async_copy(srcr.at[pl.ds(noff, SBW)],
                                      srcb.at[1 - p], isem)
                i1 = pltpu.async_copy(dstr.at[pl.ds(noff, SBW)],
                                      dstb.at[1 - p], isem)
                gd = [None] * SBW
                sd = [None] * SBW
                for k in range(SBW):
                    b = k % 2
                    if k >= 2:
                        sd[k - 2].wait()
                    gd[k] = pltpu.async_copy(
                        table.at[srcb.at[p, k]],
                        rows.at[pl.ds(b * CH, CH)], gsem)
                    if k >= 1:
                        gd[k - 1].wait()
                        sd[k - 1] = pltpu.async_copy(
                            rows.at[pl.ds(((k - 1) % 2) * CH, CH)],
                            acc.at[dstb.at[p, k - 1]], ssem, add=True)
                gd[SBW - 1].wait()
                sd[SBW - 1] = pltpu.async_copy(
                    rows.at[pl.ds(((SBW - 1) % 2) * CH, CH)],
                    acc.at[dstb.at[p, SBW - 1]], ssem, add=True)
                sd[SBW - 2].wait()
                sd[SBW - 1].wait()
                i0.wait()
                i1.wait()
            return carry

        lax.fori_loop(0, nsb // 2, outer, 0)
        plsc.subcore_barrier()

        for off, sz in chunks:
            pltpu.sync_copy(acc.at[pl.ds(s * rpt + off, sz)],
                            rows.at[pl.ds(0, sz)])

            @pl.when(c == 0)
            def _():
                pltpu.sync_copy(rows.at[pl.ds(0, sz)],
                                out0.at[pl.ds(s * rpt + off, sz)])

            @pl.when(c == 1)
            def _():
                pltpu.sync_copy(rows.at[pl.ds(0, sz)],
                                out1.at[pl.ds(s * rpt + off, sz)])

    return agg


_BR = 1000


def _mm_kernel(x_ref, w_ref, o_ref):
    o_ref[...] = jnp.dot(x_ref[...], w_ref[...],
                         preferred_element_type=jnp.float32)


def _tc_matmul(x, w):
    n, din = x.shape
    dout = w.shape[1]
    grid = (n // _BR,)
    return pl.pallas_call(
        _mm_kernel,
        grid=grid,
        in_specs=[pl.BlockSpec((_BR, din), lambda i: (i, 0)),
                  pl.BlockSpec((din, dout), lambda i: (0, 0))],
        out_specs=pl.BlockSpec((_BR, dout), lambda i: (i, 0)),
        out_shape=jax.ShapeDtypeStruct((n, dout), jnp.float32),
    )(x, w)


def _dinv_kernel(d0_ref, d1_ref, h_ref, dinv_ref, hp_ref):
    deg = d0_ref[...] + d1_ref[...] + 1.0
    dinv = 1.0 / jnp.sqrt(deg)
    dinv_ref[...] = dinv
    hp_ref[...] = h_ref[...] * dinv


def _tc_dinv_scale(deg0, deg1, h):
    n, din = h.shape
    grid = (n // _BR,)
    return pl.pallas_call(
        _dinv_kernel,
        grid=grid,
        in_specs=[pl.BlockSpec((_BR, 1), lambda i: (i, 0)),
                  pl.BlockSpec((_BR, 1), lambda i: (i, 0)),
                  pl.BlockSpec((_BR, din), lambda i: (i, 0))],
        out_specs=[pl.BlockSpec((_BR, 1), lambda i: (i, 0)),
                   pl.BlockSpec((_BR, din), lambda i: (i, 0))],
        out_shape=[jax.ShapeDtypeStruct((n, 1), jnp.float32),
                   jax.ShapeDtypeStruct((n, din), jnp.float32)],
    )(deg0, deg1, h)


_BN_SCALE = 1.0 / (1.0 + 1e-5) ** 0.5


def _mid_kernel(with_mm, *refs):
    if with_mm:
        p0_ref, p1_ref, hp_ref, dinv_ref, b_ref, g_ref, be_ref, w_ref, \
            o_ref = refs
    else:
        p0_ref, p1_ref, hp_ref, dinv_ref, b_ref, g_ref, be_ref, o_ref = refs
    dinv = dinv_ref[...]
    t = (p0_ref[...] + p1_ref[...] + hp_ref[...]) * dinv + b_ref[...]
    u = jnp.maximum(g_ref[...] * t * _BN_SCALE + be_ref[...], 0.0)
    if with_mm:
        u = jnp.dot(u, w_ref[...], preferred_element_type=jnp.float32)
    o_ref[...] = u * dinv


def _tc_mid(p0, p1, hp, dinv, b, g, be, w=None):
    n, din = hp.shape
    dout = w.shape[1] if w is not None else din
    grid = (n // _BR,)
    in_specs = [pl.BlockSpec((_BR, din), lambda i: (i, 0)),
                pl.BlockSpec((_BR, din), lambda i: (i, 0)),
                pl.BlockSpec((_BR, din), lambda i: (i, 0)),
                pl.BlockSpec((_BR, 1), lambda i: (i, 0)),
                pl.BlockSpec((1, din), lambda i: (0, 0)),
                pl.BlockSpec((1, din), lambda i: (0, 0)),
                pl.BlockSpec((1, din), lambda i: (0, 0))]
    args = [p0, p1, hp, dinv, b, g, be]
    if w is not None:
        in_specs.append(pl.BlockSpec((din, dout), lambda i: (0, 0)))
        args.append(w)
    return pl.pallas_call(
        functools.partial(_mid_kernel, w is not None),
        grid=grid,
        in_specs=in_specs,
        out_specs=pl.BlockSpec((_BR, dout), lambda i: (i, 0)),
        out_shape=jax.ShapeDtypeStruct((n, dout), jnp.float32),
    )(*args)


def _final_kernel(p0_ref, p1_ref, hp_ref, dinv_ref, w3_ref, b_ref, wv_ref,
                  bv_ref, ow_ref, ob_ref, o_ref):
    t = (p0_ref[...] + p1_ref[...] + hp_ref[...]) * dinv_ref[...]
    h = jnp.dot(t, w3_ref[...], preferred_element_type=jnp.float32) \
        + b_ref[...]
    v = jnp.dot(h, wv_ref[...], preferred_element_type=jnp.float32) \
        + bv_ref[...]
    o_ref[...] = h + jnp.dot(v, ow_ref[...],
                             preferred_element_type=jnp.float32) + ob_ref[...]


def _tc_final(p0, p1, hp, dinv, w3, b, wv_t, bv, ow_t, ob):
    n, din = hp.shape
    d = w3.shape[1]
    grid = (n // _BR,)
    return pl.pallas_call(
        _final_kernel,
        grid=grid,
        in_specs=[pl.BlockSpec((_BR, din), lambda i: (i, 0)),
                  pl.BlockSpec((_BR, din), lambda i: (i, 0)),
                  pl.BlockSpec((_BR, din), lambda i: (i, 0)),
                  pl.BlockSpec((_BR, 1), lambda i: (i, 0)),
                  pl.BlockSpec((din, d), lambda i: (0, 0)),
                  pl.BlockSpec((1, d), lambda i: (0, 0)),
                  pl.BlockSpec((d, d), lambda i: (0, 0)),
                  pl.BlockSpec((1, d), lambda i: (0, 0)),
                  pl.BlockSpec((d, d), lambda i: (0, 0)),
                  pl.BlockSpec((1, d), lambda i: (0, 0))],
        out_specs=pl.BlockSpec((_BR, d), lambda i: (i, 0)),
        out_shape=jax.ShapeDtypeStruct((n, d), jnp.float32),
    )(p0, p1, hp, dinv, w3, b, wv_t, bv, ow_t, ob)


def kernel(x, edge_index, W1, b1, g1, be1, W2, b2, g2, be2, W3, b3,
           in_w, in_b, out_w, out_b):
    n, d_in = x.shape
    e = edge_index.shape[1]
    d_out = W3.shape[1]

    tile_e = NC * NS * NSB_STEP
    e_pad = _cdiv(e, tile_e) * tile_e
    n_pad = _cdiv(n, NS * 8) * NS * 8 + NS * 8
    rpt = n_pad // NS
    assert rpt % 8 == 0

    src = edge_index[0].astype(jnp.int32)
    dst = edge_index[1].astype(jnp.int32)
    pad_i = jnp.arange(e_pad - e, dtype=jnp.int32)
    src_p = jnp.concatenate([src, pad_i % n]).reshape(e_pad // CH, CH)
    dst_p = jnp.concatenate([dst, n + pad_i % (n_pad - n)]).reshape(
        e_pad // CH, CH)

    ones_r = jnp.ones((CH,), jnp.float32)
    zer1 = jnp.zeros((rpt,), jnp.float32)
    zer_w = jnp.zeros((2 * CH, d_in), jnp.float32)

    deg0, deg1 = _make_deg(n_pad, e_pad)(dst_p, ones_r, zer1)
    deg0 = deg0[:n].reshape(n, 1)
    deg1 = deg1[:n].reshape(n, 1)

    h1 = _tc_matmul(x, W1)
    dinv, h1p = _tc_dinv_scale(deg0, deg1, h1)

    agg_w = _make_agg(n_pad, d_in, e_pad)

    p0, p1 = agg_w(h1p, src_p, dst_p, zer_w)
    h2p = _tc_mid(p0, p1, h1p, dinv, b1.reshape(1, -1), g1.reshape(1, -1),
                  be1.reshape(1, -1), W2)

    p0, p1 = agg_w(h2p, src_p, dst_p, zer_w)
    h3t = _tc_mid(p0, p1, h2p, dinv, b2.reshape(1, -1), g2.reshape(1, -1),
                  be2.reshape(1, -1))

    p0, p1 = agg_w(h3t, src_p, dst_p, zer_w)

    wv_t = in_w[2 * d_out:3 * d_out, :].T
    bv = in_b[2 * d_out:3 * d_out].reshape(1, -1)
    out = _tc_final(p0, p1, h3t, dinv, W3, b3.reshape(1, -1), wv_t, bv,
                    out_w.T, out_b.reshape(1, -1))
    return out


# speedup vs baseline: 25.2837x; 1.0327x over previous
import functools

import jax
import jax.numpy as jnp
from jax import lax
from jax.experimental import pallas as pl
from jax.experimental.pallas import tpu as pltpu
from jax.experimental.pallas import tpu_sc as plsc

NC = 2
NS = 16
CH = 128
KW = 2
WE = CH * KW
SBW = 8
NSB_STEP = 2 * SBW * CH


def _cdiv(a, b):
    return (a + b - 1) // b


def _make_deg(n_pad, e_pad):
    ept = e_pad // (NC * NS)
    n_win = ept // WE
    rpt = n_pad // NS
    mesh = plsc.VectorSubcoreMesh(core_axis_name="c", subcore_axis_name="s")

    nsb = (ept // CH) // SBW
    assert nsb % 2 == 0

    @functools.partial(
        pl.kernel,
        out_type=(jax.ShapeDtypeStruct((n_pad,), jnp.float32),
                  jax.ShapeDtypeStruct((n_pad,), jnp.float32)),
        mesh=mesh,
        scratch_types=[
            pltpu.VMEM_SHARED((n_pad,), jnp.float32),
            pltpu.VMEM((2, SBW, CH), jnp.int32),
            pltpu.VMEM((CH,), jnp.float32),
            pltpu.VMEM((rpt,), jnp.float32),
            pltpu.SemaphoreType.DMA,
            pltpu.SemaphoreType.DMA,
        ],
    )
    def deg(dstr, onesr, zerr, out0, out1, acc, dstb, onesb, stage,
            isem, ssem):
        c = lax.axis_index("c")
        s = lax.axis_index("s")
        pltpu.sync_copy(onesr, onesb)
        pltpu.sync_copy(zerr, stage)
        pltpu.sync_copy(stage, acc.at[pl.ds(s * rpt, rpt)])
        plsc.subcore_barrier()

        rbase = (c * NS + s) * (ept // CH)
        pltpu.sync_copy(dstr.at[pl.ds(rbase, SBW)], dstb.at[0])

        def outer(o, carry):
            for p in (0, 1):
                sb = 2 * o + p
                noff = rbase + lax.rem(sb + 1, nsb) * SBW
                i0 = pltpu.async_copy(dstr.at[pl.ds(noff, SBW)],
                                      dstb.at[1 - p], isem)
                sd = [pltpu.async_copy(onesb, acc.at[dstb.at[p, k]], ssem,
                                       add=True)
                      for k in range(SBW)]
                for d_ in sd:
                    d_.wait()
                i0.wait()
            return carry

        lax.fori_loop(0, nsb // 2, outer, 0)
        plsc.subcore_barrier()
        pltpu.sync_copy(acc.at[pl.ds(s * rpt, rpt)], stage)

        @pl.when(c == 0)
        def _():
            pltpu.sync_copy(stage, out0.at[pl.ds(s * rpt, rpt)])

        @pl.when(c == 1)
        def _():
            pltpu.sync_copy(stage, out1.at[pl.ds(s * rpt, rpt)])

    return deg


def _make_agg(n_pad, d, e_pad):
    ept = e_pad // (NC * NS)
    n_win = ept // CH
    nsb = n_win // SBW
    assert nsb % 2 == 0 and nsb * SBW == n_win
    rpt = n_pad // NS
    stg = 2 * CH
    chunks = [(off, min(stg, rpt - off)) for off in range(0, rpt, stg)]
    mesh = plsc.VectorSubcoreMesh(core_axis_name="c", subcore_axis_name="s")

    @functools.partial(
        pl.kernel,
        out_type=(jax.ShapeDtypeStruct((n_pad, d), jnp.float32),
                  jax.ShapeDtypeStruct((n_pad, d), jnp.float32)),
        mesh=mesh,
        scratch_types=[
            pltpu.VMEM_SHARED((n_pad, d), jnp.float32),
            pltpu.VMEM((2, SBW, CH), jnp.int32),
            pltpu.VMEM((2, SBW, CH), jnp.int32),
            pltpu.VMEM((stg, d), jnp.float32),
            pltpu.SemaphoreType.DMA,
            pltpu.SemaphoreType.DMA,
            pltpu.SemaphoreType.DMA,
        ],
    )
    def agg(table, srcr, dstr, zerr, out0, out1, acc, srcb, dstb, rows,
            isem, gsem, ssem):
        c = lax.axis_index("c")
        s = lax.axis_index("s")
        for off, sz in chunks:
            pltpu.sync_copy(zerr.at[pl.ds(0, sz)],
                            acc.at[pl.ds(s * rpt + off, sz)])
        plsc.subcore_barrier()

        rbase = (c * NS + s) * n_win

        pltpu.sync_copy(srcr.at[pl.ds(rbase, SBW)], srcb.at[0])
        pltpu.sync_copy(dstr.at[pl.ds(rbase, SBW)], dstb.at[0])

        def outer(o, carry):
            for p in (0, 1):
                sb = 2 * o + p
                nxt = lax.rem(sb + 1, nsb)
                noff = rbase + nxt * SBW
                i0 = pltpu.async_copy(srcr.at[pl.ds(noff, SBW)],
                                      srcb.at[1 - p], isem)
                i1 = pltpu.async_copy(dstr.at[pl.ds(noff, SBW)],
                                      dstb.at[1 - p], isem)
                gd = [None] * SBW
                sd = [None] * SBW
                for k in range(SBW):
                    b = k % 2
                    if k >= 2:
                        sd[k - 2].wait()
                    gd[k] = pltpu.async_copy(
                        table.at[srcb.at[p, k]],
                        rows.at[pl.ds(b * CH, CH)], gsem)
                    if k >= 1:
                        gd[k - 1].wait()
                        sd[k - 1] = pltpu.async_copy(
                            rows.at[pl.ds(((k - 1) % 2) * CH, CH)],
                            acc.at[dstb.at[p, k - 1]], ssem, add=True)
                gd[SBW - 1].wait()
                sd[SBW - 1] = pltpu.async_copy(
                    rows.at[pl.ds(((SBW - 1) % 2) * CH, CH)],
                    acc.at[dstb.at[p, SBW - 1]], ssem, add=True)
                sd[SBW - 2].wait()
                sd[SBW - 1].wait()
                i0.wait()
                i1.wait()
            return carry

        lax.fori_loop(0, nsb // 2, outer, 0)
        plsc.subcore_barrier()

        @pl.when(c == 0)
        def _():
            pltpu.sync_copy(acc.at[pl.ds(s * rpt, rpt)],
                            out0.at[pl.ds(s * rpt, rpt)])

        @pl.when(c == 1)
        def _():
            pltpu.sync_copy(acc.at[pl.ds(s * rpt, rpt)],
                            out1.at[pl.ds(s * rpt, rpt)])

    return agg


_BR = 1000


def _mm_kernel(x_ref, w_ref, o_ref):
    o_ref[...] = jnp.dot(x_ref[...], w_ref[...],
                         preferred_element_type=jnp.float32)


def _tc_matmul(x, w):
    n, din = x.shape
    dout = w.shape[1]
    grid = (n // _BR,)
    return pl.pallas_call(
        _mm_kernel,
        grid=grid,
        in_specs=[pl.BlockSpec((_BR, din), lambda i: (i, 0)),
                  pl.BlockSpec((din, dout), lambda i: (0, 0))],
        out_specs=pl.BlockSpec((_BR, dout), lambda i: (i, 0)),
        out_shape=jax.ShapeDtypeStruct((n, dout), jnp.float32),
    )(x, w)


def _dinv_kernel(d0_ref, d1_ref, h_ref, dinv_ref, hp_ref):
    deg = d0_ref[...] + d1_ref[...] + 1.0
    dinv = 1.0 / jnp.sqrt(deg)
    dinv_ref[...] = dinv
    hp_ref[...] = h_ref[...] * dinv


def _tc_dinv_scale(deg0, deg1, h):
    n, din = h.shape
    grid = (n // _BR,)
    return pl.pallas_call(
        _dinv_kernel,
        grid=grid,
        in_specs=[pl.BlockSpec((_BR, 1), lambda i: (i, 0)),
                  pl.BlockSpec((_BR, 1), lambda i: (i, 0)),
                  pl.BlockSpec((_BR, din), lambda i: (i, 0))],
        out_specs=[pl.BlockSpec((_BR, 1), lambda i: (i, 0)),
                   pl.BlockSpec((_BR, din), lambda i: (i, 0))],
        out_shape=[jax.ShapeDtypeStruct((n, 1), jnp.float32),
                   jax.ShapeDtypeStruct((n, din), jnp.float32)],
    )(deg0, deg1, h)


_BN_SCALE = 1.0 / (1.0 + 1e-5) ** 0.5


def _mid_kernel(with_mm, *refs):
    if with_mm:
        p0_ref, p1_ref, hp_ref, dinv_ref, b_ref, g_ref, be_ref, w_ref, \
            o_ref = refs
    else:
        p0_ref, p1_ref, hp_ref, dinv_ref, b_ref, g_ref, be_ref, o_ref = refs
    dinv = dinv_ref[...]
    t = (p0_ref[...] + p1_ref[...] + hp_ref[...]) * dinv + b_ref[...]
    u = jnp.maximum(g_ref[...] * t * _BN_SCALE + be_ref[...], 0.0)
    if with_mm:
        u = jnp.dot(u, w_ref[...], preferred_element_type=jnp.float32)
    o_ref[...] = u * dinv


def _tc_mid(p0, p1, hp, dinv, b, g, be, w=None):
    n, din = hp.shape
    dout = w.shape[1] if w is not None else din
    grid = (n // _BR,)
    in_specs = [pl.BlockSpec((_BR, din), lambda i: (i, 0)),
                pl.BlockSpec((_BR, din), lambda i: (i, 0)),
                pl.BlockSpec((_BR, din), lambda i: (i, 0)),
                pl.BlockSpec((_BR, 1), lambda i: (i, 0)),
                pl.BlockSpec((1, din), lambda i: (0, 0)),
                pl.BlockSpec((1, din), lambda i: (0, 0)),
                pl.BlockSpec((1, din), lambda i: (0, 0))]
    args = [p0, p1, hp, dinv, b, g, be]
    if w is not None:
        in_specs.append(pl.BlockSpec((din, dout), lambda i: (0, 0)))
        args.append(w)
    return pl.pallas_call(
        functools.partial(_mid_kernel, w is not None),
        grid=grid,
        in_specs=in_specs,
        out_specs=pl.BlockSpec((_BR, dout), lambda i: (i, 0)),
        out_shape=jax.ShapeDtypeStruct((n, dout), jnp.float32),
    )(*args)


def _final_kernel(p0_ref, p1_ref, hp_ref, dinv_ref, w3_ref, b_ref, wv_ref,
                  bv_ref, ow_ref, ob_ref, o_ref):
    t = (p0_ref[...] + p1_ref[...] + hp_ref[...]) * dinv_ref[...]
    h = jnp.dot(t, w3_ref[...], preferred_element_type=jnp.float32) \
        + b_ref[...]
    v = jnp.dot(h, wv_ref[...], preferred_element_type=jnp.float32) \
        + bv_ref[...]
    o_ref[...] = h + jnp.dot(v, ow_ref[...],
                             preferred_element_type=jnp.float32) + ob_ref[...]


def _tc_final(p0, p1, hp, dinv, w3, b, wv_t, bv, ow_t, ob):
    n, din = hp.shape
    d = w3.shape[1]
    grid = (n // _BR,)
    return pl.pallas_call(
        _final_kernel,
        grid=grid,
        in_specs=[pl.BlockSpec((_BR, din), lambda i: (i, 0)),
                  pl.BlockSpec((_BR, din), lambda i: (i, 0)),
                  pl.BlockSpec((_BR, din), lambda i: (i, 0)),
                  pl.BlockSpec((_BR, 1), lambda i: (i, 0)),
                  pl.BlockSpec((din, d), lambda i: (0, 0)),
                  pl.BlockSpec((1, d), lambda i: (0, 0)),
                  pl.BlockSpec((d, d), lambda i: (0, 0)),
                  pl.BlockSpec((1, d), lambda i: (0, 0)),
                  pl.BlockSpec((d, d), lambda i: (0, 0)),
                  pl.BlockSpec((1, d), lambda i: (0, 0))],
        out_specs=pl.BlockSpec((_BR, d), lambda i: (i, 0)),
        out_shape=jax.ShapeDtypeStruct((n, d), jnp.float32),
    )(p0, p1, hp, dinv, w3, b, wv_t, bv, ow_t, ob)


def kernel(x, edge_index, W1, b1, g1, be1, W2, b2, g2, be2, W3, b3,
           in_w, in_b, out_w, out_b):
    n, d_in = x.shape
    e = edge_index.shape[1]
    d_out = W3.shape[1]

    tile_e = NC * NS * NSB_STEP
    e_pad = _cdiv(e, tile_e) * tile_e
    n_pad = _cdiv(n, NS * 8) * NS * 8 + NS * 8
    rpt = n_pad // NS
    assert rpt % 8 == 0

    src = edge_index[0].astype(jnp.int32)
    dst = edge_index[1].astype(jnp.int32)
    pad_i = jnp.arange(e_pad - e, dtype=jnp.int32)
    src_p = jnp.concatenate([src, pad_i % n]).reshape(e_pad // CH, CH)
    dst_p = jnp.concatenate([dst, n + pad_i % (n_pad - n)]).reshape(
        e_pad // CH, CH)

    ones_r = jnp.ones((CH,), jnp.float32)
    zer1 = jnp.zeros((rpt,), jnp.float32)
    zer_w = jnp.zeros((2 * CH, d_in), jnp.float32)

    deg0, deg1 = _make_deg(n_pad, e_pad)(dst_p, ones_r, zer1)
    deg0 = deg0[:n].reshape(n, 1)
    deg1 = deg1[:n].reshape(n, 1)

    h1 = _tc_matmul(x, W1)
    dinv, h1p = _tc_dinv_scale(deg0, deg1, h1)

    agg_w = _make_agg(n_pad, d_in, e_pad)

    p0, p1 = agg_w(h1p, src_p, dst_p, zer_w)
    h2p = _tc_mid(p0, p1, h1p, dinv, b1.reshape(1, -1), g1.reshape(1, -1),
                  be1.reshape(1, -1), W2)

    p0, p1 = agg_w(h2p, src_p, dst_p, zer_w)
    h3t = _tc_mid(p0, p1, h2p, dinv, b2.reshape(1, -1), g2.reshape(1, -1),
                  be2.reshape(1, -1))

    p0, p1 = agg_w(h3t, src_p, dst_p, zer_w)

    wv_t = in_w[2 * d_out:3 * d_out, :].T
    bv = in_b[2 * d_out:3 * d_out].reshape(1, -1)
    out = _tc_final(p0, p1, h3t, dinv, W3, b3.reshape(1, -1), wv_t, bv,
                    out_w.T, out_b.reshape(1, -1))
    return out


# speedup vs baseline: 26.0372x; 1.0298x over previous
import functools

import jax
import jax.numpy as jnp
from jax import lax
from jax.experimental import pallas as pl
from jax.experimental.pallas import tpu as pltpu
from jax.experimental.pallas import tpu_sc as plsc

NC = 2
NS = 16
CH = 128
KW = 2
WE = CH * KW
SBW = 8
NSB_STEP = 2 * SBW * CH


def _cdiv(a, b):
    return (a + b - 1) // b


def _make_deg(n_pad, e_pad):
    ept = e_pad // (NC * NS)
    n_win = ept // WE
    rpt = n_pad // NS
    mesh = plsc.VectorSubcoreMesh(core_axis_name="c", subcore_axis_name="s")

    nsb = (ept // CH) // SBW
    assert nsb % 2 == 0

    @functools.partial(
        pl.kernel,
        out_type=(jax.ShapeDtypeStruct((n_pad,), jnp.float32),
                  jax.ShapeDtypeStruct((n_pad,), jnp.float32)),
        mesh=mesh,
        scratch_types=[
            pltpu.VMEM_SHARED((n_pad,), jnp.float32),
            pltpu.VMEM((2, SBW, CH), jnp.int32),
            pltpu.VMEM((CH,), jnp.float32),
            pltpu.VMEM((rpt,), jnp.float32),
            pltpu.SemaphoreType.DMA,
            pltpu.SemaphoreType.DMA,
        ],
    )
    def deg(dstr, onesr, zerr, out0, out1, acc, dstb, onesb, stage,
            isem, ssem):
        c = lax.axis_index("c")
        s = lax.axis_index("s")
        pltpu.sync_copy(onesr, onesb)
        pltpu.sync_copy(zerr, stage)
        pltpu.sync_copy(stage, acc.at[pl.ds(s * rpt, rpt)])
        plsc.subcore_barrier()

        rbase = (c * NS + s) * (ept // CH)
        pltpu.sync_copy(dstr.at[pl.ds(rbase, SBW)], dstb.at[0])

        def outer(o, carry):
            for p in (0, 1):
                sb = 2 * o + p
                noff = rbase + lax.rem(sb + 1, nsb) * SBW
                i0 = pltpu.async_copy(dstr.at[pl.ds(noff, SBW)],
                                      dstb.at[1 - p], isem)
                sd = [pltpu.async_copy(onesb, acc.at[dstb.at[p, k]], ssem,
                                       add=True)
                      for k in range(SBW)]
                for d_ in sd:
                    d_.wait()
                i0.wait()
            return carry

        lax.fori_loop(0, nsb // 2, outer, 0)
        plsc.subcore_barrier()
        pltpu.sync_copy(acc.at[pl.ds(s * rpt, rpt)], stage)

        @pl.when(c == 0)
        def _():
            pltpu.sync_copy(stage, out0.at[pl.ds(s * rpt, rpt)])

        @pl.when(c == 1)
        def _():
            pltpu.sync_copy(stage, out1.at[pl.ds(s * rpt, rpt)])

    return deg


def _make_agg(n, n_pad, d, e_pad):
    ept = e_pad // (NC * NS)
    n_win = ept // CH
    nsb = n_win // SBW
    assert nsb % 2 == 0 and nsb * SBW == n_win
    rpt = n_pad // NS
    stg = 2 * CH
    chunks = [(off, min(stg, rpt - off)) for off in range(0, rpt, stg)]
    mesh = plsc.VectorSubcoreMesh(core_axis_name="c", subcore_axis_name="s")

    @functools.partial(
        pl.kernel,
        out_type=(jax.ShapeDtypeStruct((n_pad, d), jnp.float32),
                  jax.ShapeDtypeStruct((n_pad, d), jnp.float32)),
        mesh=mesh,
        scratch_types=[
            pltpu.VMEM_SHARED((n_pad, d), jnp.float32),
            pltpu.VMEM((2, SBW, CH), jnp.int32),
            pltpu.VMEM((2, SBW, CH), jnp.int32),
            pltpu.VMEM((stg, d), jnp.float32),
            pltpu.SemaphoreType.DMA,
            pltpu.SemaphoreType.DMA,
            pltpu.SemaphoreType.DMA,
        ],
    )
    def agg(table, srcr, dstr, zerr, out0, out1, acc, srcb, dstb, rows,
            isem, gsem, ssem):
        c = lax.axis_index("c")
        s = lax.axis_index("s")
        last_full = n // rpt
        rem_rows = n - last_full * rpt
        pad_rows = n_pad - n
        assert rem_rows % 8 == 0 and pad_rows % 8 == 0 and pad_rows <= 2 * CH

        @pl.when(c == 0)
        def _():
            @pl.when(s < last_full)
            def _():
                pltpu.sync_copy(table.at[pl.ds(s * rpt, rpt)],
                                acc.at[pl.ds(s * rpt, rpt)])

            @pl.when(s == last_full)
            def _():
                if rem_rows:
                    pltpu.sync_copy(table.at[pl.ds(last_full * rpt,
                                                   rem_rows)],
                                    acc.at[pl.ds(last_full * rpt,
                                                 rem_rows)])
                pltpu.sync_copy(zerr.at[pl.ds(0, pad_rows)],
                                acc.at[pl.ds(n, pad_rows)])

        @pl.when(c == 1)
        def _():
            for off, sz in chunks:
                pltpu.sync_copy(zerr.at[pl.ds(0, sz)],
                                acc.at[pl.ds(s * rpt + off, sz)])
        plsc.subcore_barrier()

        rbase = (c * NS + s) * n_win

        pltpu.sync_copy(srcr.at[pl.ds(rbase, SBW)], srcb.at[0])
        pltpu.sync_copy(dstr.at[pl.ds(rbase, SBW)], dstb.at[0])

        def outer(o, carry):
            for p in (0, 1):
                sb = 2 * o + p
                nxt = lax.rem(sb + 1, nsb)
                noff = rbase + nxt * SBW
                i0 = pltpu.async_copy(srcr.at[pl.ds(noff, SBW)],
                                      srcb.at[1 - p], isem)
                i1 = pltpu.async_copy(dstr.at[pl.ds(noff, SBW)],
                                      dstb.at[1 - p], isem)
                gd = [None] * SBW
                sd = [None] * SBW
                for k in range(SBW):
                    b = k % 2
                    if k >= 2:
                        sd[k - 2].wait()
                    gd[k] = pltpu.async_copy(
                        table.at[srcb.at[p, k]],
                        rows.at[pl.ds(b * CH, CH)], gsem)
                    if k >= 1:
                        gd[k - 1].wait()
                        sd[k - 1] = pltpu.async_copy(
                            rows.at[pl.ds(((k - 1) % 2) * CH, CH)],
                            acc.at[dstb.at[p, k - 1]], ssem, add=True)
                gd[SBW - 1].wait()
                sd[SBW - 1] = pltpu.async_copy(
                    rows.at[pl.ds(((SBW - 1) % 2) * CH, CH)],
                    acc.at[dstb.at[p, SBW - 1]], ssem, add=True)
                sd[SBW - 2].wait()
                sd[SBW - 1].wait()
                i0.wait()
                i1.wait()
            return carry

        lax.fori_loop(0, nsb // 2, outer, 0)
        plsc.subcore_barrier()

        @pl.when(c == 0)
        def _():
            pltpu.sync_copy(acc.at[pl.ds(s * rpt, rpt)],
                            out0.at[pl.ds(s * rpt, rpt)])

        @pl.when(c == 1)
        def _():
            pltpu.sync_copy(acc.at[pl.ds(s * rpt, rpt)],
                            out1.at[pl.ds(s * rpt, rpt)])

    return agg


_BR = 1000


def _mm_kernel(x_ref, w_ref, d0_ref, d1_ref, dinv_ref, hp_ref):
    dinv = 1.0 / jnp.sqrt(d0_ref[...] + d1_ref[...] + 1.0)
    dinv_ref[...] = dinv
    hp_ref[...] = jnp.dot(x_ref[...], w_ref[...],
                          preferred_element_type=jnp.float32) * dinv


def _tc_mm_scale(x, w, deg0, deg1):
    n, din = x.shape
    dout = w.shape[1]
    grid = (n // _BR,)
    return pl.pallas_call(
        _mm_kernel,
        grid=grid,
        in_specs=[pl.BlockSpec((_BR, din), lambda i: (i, 0)),
                  pl.BlockSpec((din, dout), lambda i: (0, 0)),
                  pl.BlockSpec((_BR, 1), lambda i: (i, 0)),
                  pl.BlockSpec((_BR, 1), lambda i: (i, 0))],
        out_specs=[pl.BlockSpec((_BR, 1), lambda i: (i, 0)),
                   pl.BlockSpec((_BR, dout), lambda i: (i, 0))],
        out_shape=[jax.ShapeDtypeStruct((n, 1), jnp.float32),
                   jax.ShapeDtypeStruct((n, dout), jnp.float32)],
    )(x, w, deg0, deg1)


_BN_SCALE = 1.0 / (1.0 + 1e-5) ** 0.5


def _mid_kernel(with_mm, *refs):
    if with_mm:
        p0_ref, p1_ref, dinv_ref, b_ref, g_ref, be_ref, w_ref, o_ref = refs
    else:
        p0_ref, p1_ref, dinv_ref, b_ref, g_ref, be_ref, o_ref = refs
    dinv = dinv_ref[...]
    t = (p0_ref[...] + p1_ref[...]) * dinv + b_ref[...]
    u = jnp.maximum(g_ref[...] * t * _BN_SCALE + be_ref[...], 0.0)
    if with_mm:
        u = jnp.dot(u, w_ref[...], preferred_element_type=jnp.float32)
    o_ref[...] = u * dinv


def _tc_mid(p0, p1, dinv, b, g, be, w=None, din=None):
    n = dinv.shape[0]
    dout = w.shape[1] if w is not None else din
    grid = (n // _BR,)
    in_specs = [pl.BlockSpec((_BR, din), lambda i: (i, 0)),
                pl.BlockSpec((_BR, din), lambda i: (i, 0)),
                pl.BlockSpec((_BR, 1), lambda i: (i, 0)),
                pl.BlockSpec((1, din), lambda i: (0, 0)),
                pl.BlockSpec((1, din), lambda i: (0, 0)),
                pl.BlockSpec((1, din), lambda i: (0, 0))]
    args = [p0, p1, dinv, b, g, be]
    if w is not None:
        in_specs.append(pl.BlockSpec((din, dout), lambda i: (0, 0)))
        args.append(w)
    return pl.pallas_call(
        functools.partial(_mid_kernel, w is not None),
        grid=grid,
        in_specs=in_specs,
        out_specs=pl.BlockSpec((_BR, dout), lambda i: (i, 0)),
        out_shape=jax.ShapeDtypeStruct((n, dout), jnp.float32),
    )(*args)


def _final_kernel(p0_ref, p1_ref, dinv_ref, w3_ref, b_ref, wv_ref,
                  bv_ref, ow_ref, ob_ref, o_ref):
    t = (p0_ref[...] + p1_ref[...]) * dinv_ref[...]
    h = jnp.dot(t, w3_ref[...], preferred_element_type=jnp.float32) \
        + b_ref[...]
    v = jnp.dot(h, wv_ref[...], preferred_element_type=jnp.float32) \
        + bv_ref[...]
    o_ref[...] = h + jnp.dot(v, ow_ref[...],
                             preferred_element_type=jnp.float32) + ob_ref[...]


def _tc_final(p0, p1, dinv, w3, b, wv_t, bv, ow_t, ob):
    n = dinv.shape[0]
    din, d = w3.shape
    grid = (n // _BR,)
    return pl.pallas_call(
        _final_kernel,
        grid=grid,
        in_specs=[pl.BlockSpec((_BR, din), lambda i: (i, 0)),
                  pl.BlockSpec((_BR, din), lambda i: (i, 0)),
                  pl.BlockSpec((_BR, 1), lambda i: (i, 0)),
                  pl.BlockSpec((din, d), lambda i: (0, 0)),
                  pl.BlockSpec((1, d), lambda i: (0, 0)),
                  pl.BlockSpec((d, d), lambda i: (0, 0)),
                  pl.BlockSpec((1, d), lambda i: (0, 0)),
                  pl.BlockSpec((d, d), lambda i: (0, 0)),
                  pl.BlockSpec((1, d), lambda i: (0, 0))],
        out_specs=pl.BlockSpec((_BR, d), lambda i: (i, 0)),
        out_shape=jax.ShapeDtypeStruct((n, d), jnp.float32),
    )(p0, p1, dinv, w3, b, wv_t, bv, ow_t, ob)


def kernel(x, edge_index, W1, b1, g1, be1, W2, b2, g2, be2, W3, b3,
           in_w, in_b, out_w, out_b):
    n, d_in = x.shape
    e = edge_index.shape[1]
    d_out = W3.shape[1]

    tile_e = NC * NS * NSB_STEP
    e_pad = _cdiv(e, tile_e) * tile_e
    n_pad = _cdiv(n, NS * 8) * NS * 8 + NS * 8
    rpt = n_pad // NS
    assert rpt % 8 == 0

    src = edge_index[0].astype(jnp.int32)
    dst = edge_index[1].astype(jnp.int32)
    pad_i = jnp.arange(e_pad - e, dtype=jnp.int32)
    src_p = jnp.concatenate([src, pad_i % n]).reshape(e_pad // CH, CH)
    dst_p = jnp.concatenate([dst, n + pad_i % (n_pad - n)]).reshape(
        e_pad // CH, CH)

    ones_r = jnp.ones((CH,), jnp.float32)
    zer1 = jnp.zeros((rpt,), jnp.float32)
    zer_w = jnp.zeros((2 * CH, d_in), jnp.float32)

    deg0, deg1 = _make_deg(n_pad, e_pad)(dst_p, ones_r, zer1)
    deg0 = deg0[:n].reshape(n, 1)
    deg1 = deg1[:n].reshape(n, 1)

    dinv, h1p = _tc_mm_scale(x, W1, deg0, deg1)

    agg_w = _make_agg(n, n_pad, d_in, e_pad)

    p0, p1 = agg_w(h1p, src_p, dst_p, zer_w)
    h2p = _tc_mid(p0, p1, dinv, b1.reshape(1, -1), g1.reshape(1, -1),
                  be1.reshape(1, -1), W2, din=d_in)

    p0, p1 = agg_w(h2p, src_p, dst_p, zer_w)
    h3t = _tc_mid(p0, p1, dinv, b2.reshape(1, -1), g2.reshape(1, -1),
                  be2.reshape(1, -1), din=d_in)

    p0, p1 = agg_w(h3t, src_p, dst_p, zer_w)

    wv_t = in_w[2 * d_out:3 * d_out, :].T
    bv = in_b[2 * d_out:3 * d_out].reshape(1, -1)
    out = _tc_final(p0, p1, dinv, W3, b3.reshape(1, -1), wv_t, bv,
                    out_w.T, out_b.reshape(1, -1))
    return out


# speedup vs baseline: 26.2917x; 1.0098x over previous
import functools

import jax
import jax.numpy as jnp
from jax import lax
from jax.experimental import pallas as pl
from jax.experimental.pallas import tpu as pltpu
from jax.experimental.pallas import tpu_sc as plsc

NC = 2
NS = 16
CH = 128
KW = 2
WE = CH * KW
SBW = 8
NSB_STEP = 2 * SBW * CH


def _cdiv(a, b):
    return (a + b - 1) // b


def _make_deg(n_pad, e_pad):
    ept = e_pad // (NC * NS)
    n_win = ept // WE
    rpt = n_pad // NS
    mesh = plsc.VectorSubcoreMesh(core_axis_name="c", subcore_axis_name="s")

    nsb = (ept // CH) // SBW
    assert nsb % 2 == 0

    @functools.partial(
        pl.kernel,
        out_type=(jax.ShapeDtypeStruct((n_pad,), jnp.float32),
                  jax.ShapeDtypeStruct((n_pad,), jnp.float32)),
        mesh=mesh,
        scratch_types=[
            pltpu.VMEM_SHARED((n_pad,), jnp.float32),
            pltpu.VMEM((2, SBW, CH), jnp.int32),
            pltpu.VMEM((CH,), jnp.float32),
            pltpu.VMEM((rpt,), jnp.float32),
            pltpu.SemaphoreType.DMA,
            pltpu.SemaphoreType.DMA,
        ],
    )
    def deg(dstr, onesr, zerr, out0, out1, acc, dstb, onesb, stage,
            isem, ssem):
        c = lax.axis_index("c")
        s = lax.axis_index("s")
        pltpu.sync_copy(onesr, onesb)
        pltpu.sync_copy(zerr, stage)
        pltpu.sync_copy(stage, acc.at[pl.ds(s * rpt, rpt)])
        plsc.subcore_barrier()

        rbase = (c * NS + s) * (ept // CH)
        pltpu.sync_copy(dstr.at[pl.ds(rbase, SBW)], dstb.at[0])

        def outer(o, carry):
            for p in (0, 1):
                sb = 2 * o + p
                noff = rbase + lax.rem(sb + 1, nsb) * SBW
                i0 = pltpu.async_copy(dstr.at[pl.ds(noff, SBW)],
                                      dstb.at[1 - p], isem)
                sd = [pltpu.async_copy(onesb, acc.at[dstb.at[p, k]], ssem,
                                       add=True)
                      for k in range(SBW)]
                for d_ in sd:
                    d_.wait()
                i0.wait()
            return carry

        lax.fori_loop(0, nsb // 2, outer, 0)
        plsc.subcore_barrier()
        pltpu.sync_copy(acc.at[pl.ds(s * rpt, rpt)], stage)

        @pl.when(c == 0)
        def _():
            pltpu.sync_copy(stage, out0.at[pl.ds(s * rpt, rpt)])

        @pl.when(c == 1)
        def _():
            pltpu.sync_copy(stage, out1.at[pl.ds(s * rpt, rpt)])

    return deg


def _make_agg(n, n_pad, d, e_pad):
    ept = e_pad // (NC * NS)
    n_win = ept // CH
    nsb = n_win // SBW
    assert nsb % 2 == 0 and nsb * SBW == n_win
    rpt = n_pad // NS
    stg = 2 * CH
    chunks = [(off, min(stg, rpt - off)) for off in range(0, rpt, stg)]
    mesh = plsc.VectorSubcoreMesh(core_axis_name="c", subcore_axis_name="s")

    @functools.partial(
        pl.kernel,
        out_type=(jax.ShapeDtypeStruct((n_pad, d), jnp.float32),
                  jax.ShapeDtypeStruct((n_pad, d), jnp.float32)),
        mesh=mesh,
        scratch_types=[
            pltpu.VMEM_SHARED((n_pad, d), jnp.float32),
            pltpu.VMEM((2, SBW, CH), jnp.int32),
            pltpu.VMEM((2, SBW, CH), jnp.int32),
            pltpu.VMEM((stg, d), jnp.float32),
            pltpu.SemaphoreType.DMA,
            pltpu.SemaphoreType.DMA,
            pltpu.SemaphoreType.DMA,
        ],
    )
    def agg(table, srcr, dstr, zerr, out0, out1, acc, srcb, dstb, rows,
            isem, gsem, ssem):
        c = lax.axis_index("c")
        s = lax.axis_index("s")
        last_full = n // rpt
        rem_rows = n - last_full * rpt
        pad_rows = n_pad - n
        assert rem_rows % 8 == 0 and pad_rows % 8 == 0 and pad_rows <= 2 * CH

        rbase = (c * NS + s) * n_win
        p0d = pltpu.async_copy(srcr.at[pl.ds(rbase, SBW)], srcb.at[0], isem)
        p1d = pltpu.async_copy(dstr.at[pl.ds(rbase, SBW)], dstb.at[0], isem)

        @pl.when(c == 0)
        def _():
            @pl.when(s < last_full)
            def _():
                pltpu.async_copy(table.at[pl.ds(s * rpt, rpt)],
                                 acc.at[pl.ds(s * rpt, rpt)], gsem).wait()

            @pl.when(s == last_full)
            def _():
                if rem_rows:
                    d0 = pltpu.async_copy(
                        table.at[pl.ds(last_full * rpt, rem_rows)],
                        acc.at[pl.ds(last_full * rpt, rem_rows)], gsem)
                d1 = pltpu.async_copy(zerr.at[pl.ds(0, pad_rows)],
                                      acc.at[pl.ds(n, pad_rows)], gsem)
                if rem_rows:
                    d0.wait()
                d1.wait()

        @pl.when(c == 1)
        def _():
            ds_ = [pltpu.async_copy(zerr.at[pl.ds(0, sz)],
                                    acc.at[pl.ds(s * rpt + off, sz)], gsem)
                   for off, sz in chunks]
            for d_ in ds_:
                d_.wait()
        p0d.wait()
        p1d.wait()
        plsc.subcore_barrier()

        def outer(o, carry):
            for p in (0, 1):
                sb = 2 * o + p
                nxt = lax.rem(sb + 1, nsb)
                noff = rbase + nxt * SBW
                i0 = pltpu.async_copy(srcr.at[pl.ds(noff, SBW)],
                                      srcb.at[1 - p], isem)
                i1 = pltpu.async_copy(dstr.at[pl.ds(noff, SBW)],
                                      dstb.at[1 - p], isem)
                gd = [None] * SBW
                sd = [None] * SBW
                for k in range(SBW):
                    b = k % 2
                    if k >= 2:
                        sd[k - 2].wait()
                    gd[k] = pltpu.async_copy(
                        table.at[srcb.at[p, k]],
                        rows.at[pl.ds(b * CH, CH)], gsem)
                    if k >= 1:
                        gd[k - 1].wait()
                        sd[k - 1] = pltpu.async_copy(
                            rows.at[pl.ds(((k - 1) % 2) * CH, CH)],
                            acc.at[dstb.at[p, k - 1]], ssem, add=True)
                gd[SBW - 1].wait()
                sd[SBW - 1] = pltpu.async_copy(
                    rows.at[pl.ds(((SBW - 1) % 2) * CH, CH)],
                    acc.at[dstb.at[p, SBW - 1]], ssem, add=True)
                sd[SBW - 2].wait()
                sd[SBW - 1].wait()
                i0.wait()
                i1.wait()
            return carry

        lax.fori_loop(0, nsb // 2, outer, 0)
        plsc.subcore_barrier()

        @pl.when(c == 0)
        def _():
            pltpu.sync_copy(acc.at[pl.ds(s * rpt, rpt)],
                            out0.at[pl.ds(s * rpt, rpt)])

        @pl.when(c == 1)
        def _():
            pltpu.sync_copy(acc.at[pl.ds(s * rpt, rpt)],
                            out1.at[pl.ds(s * rpt, rpt)])

    return agg


_BR = 1000


def _mm_kernel(x_ref, w_ref, d0_ref, d1_ref, dinv_ref, hp_ref):
    dinv = 1.0 / jnp.sqrt(d0_ref[...] + d1_ref[...] + 1.0)
    dinv_ref[...] = dinv
    hp_ref[...] = jnp.dot(x_ref[...], w_ref[...],
                          preferred_element_type=jnp.float32) * dinv


def _tc_mm_scale(x, w, deg0, deg1):
    n, din = x.shape
    dout = w.shape[1]
    grid = (n // _BR,)
    return pl.pallas_call(
        _mm_kernel,
        grid=grid,
        in_specs=[pl.BlockSpec((_BR, din), lambda i: (i, 0)),
                  pl.BlockSpec((din, dout), lambda i: (0, 0)),
                  pl.BlockSpec((_BR, 1), lambda i: (i, 0)),
                  pl.BlockSpec((_BR, 1), lambda i: (i, 0))],
        out_specs=[pl.BlockSpec((_BR, 1), lambda i: (i, 0)),
                   pl.BlockSpec((_BR, dout), lambda i: (i, 0))],
        out_shape=[jax.ShapeDtypeStruct((n, 1), jnp.float32),
                   jax.ShapeDtypeStruct((n, dout), jnp.float32)],
    )(x, w, deg0, deg1)


_BN_SCALE = 1.0 / (1.0 + 1e-5) ** 0.5


def _mid_kernel(with_mm, *refs):
    if with_mm:
        p0_ref, p1_ref, dinv_ref, b_ref, g_ref, be_ref, w_ref, o_ref = refs
    else:
        p0_ref, p1_ref, dinv_ref, b_ref, g_ref, be_ref, o_ref = refs
    dinv = dinv_ref[...]
    t = (p0_ref[...] + p1_ref[...]) * dinv + b_ref[...]
    u = jnp.maximum(g_ref[...] * t * _BN_SCALE + be_ref[...], 0.0)
    if with_mm:
        u = jnp.dot(u, w_ref[...], preferred_element_type=jnp.float32)
    o_ref[...] = u * dinv


def _tc_mid(p0, p1, dinv, b, g, be, w=None, din=None):
    n = dinv.shape[0]
    dout = w.shape[1] if w is not None else din
    grid = (n // _BR,)
    in_specs = [pl.BlockSpec((_BR, din), lambda i: (i, 0)),
                pl.BlockSpec((_BR, din), lambda i: (i, 0)),
                pl.BlockSpec((_BR, 1), lambda i: (i, 0)),
                pl.BlockSpec((1, din), lambda i: (0, 0)),
                pl.BlockSpec((1, din), lambda i: (0, 0)),
                pl.BlockSpec((1, din), lambda i: (0, 0))]
    args = [p0, p1, dinv, b, g, be]
    if w is not None:
        in_specs.append(pl.BlockSpec((din, dout), lambda i: (0, 0)))
        args.append(w)
    return pl.pallas_call(
        functools.partial(_mid_kernel, w is not None),
        grid=grid,
        in_specs=in_specs,
        out_specs=pl.BlockSpec((_BR, dout), lambda i: (i, 0)),
        out_shape=jax.ShapeDtypeStruct((n, dout), jnp.float32),
    )(*args)


def _final_kernel(p0_ref, p1_ref, dinv_ref, w3_ref, b_ref, wv_ref,
                  bv_ref, ow_ref, ob_ref, o_ref):
    t = (p0_ref[...] + p1_ref[...]) * dinv_ref[...]
    h = jnp.dot(t, w3_ref[...], preferred_element_type=jnp.float32) \
        + b_ref[...]
    v = jnp.dot(h, wv_ref[...], preferred_element_type=jnp.float32) \
        + bv_ref[...]
    o_ref[...] = h + jnp.dot(v, ow_ref[...],
                             preferred_element_type=jnp.float32) + ob_ref[...]


def _tc_final(p0, p1, dinv, w3, b, wv_t, bv, ow_t, ob):
    n = dinv.shape[0]
    din, d = w3.shape
    grid = (n // _BR,)
    return pl.pallas_call(
        _final_kernel,
        grid=grid,
        in_specs=[pl.BlockSpec((_BR, din), lambda i: (i, 0)),
                  pl.BlockSpec((_BR, din), lambda i: (i, 0)),
                  pl.BlockSpec((_BR, 1), lambda i: (i, 0)),
                  pl.BlockSpec((din, d), lambda i: (0, 0)),
                  pl.BlockSpec((1, d), lambda i: (0, 0)),
                  pl.BlockSpec((d, d), lambda i: (0, 0)),
                  pl.BlockSpec((1, d), lambda i: (0, 0)),
                  pl.BlockSpec((d, d), lambda i: (0, 0)),
                  pl.BlockSpec((1, d), lambda i: (0, 0))],
        out_specs=pl.BlockSpec((_BR, d), lambda i: (i, 0)),
        out_shape=jax.ShapeDtypeStruct((n, d), jnp.float32),
    )(p0, p1, dinv, w3, b, wv_t, bv, ow_t, ob)


def kernel(x, edge_index, W1, b1, g1, be1, W2, b2, g2, be2, W3, b3,
           in_w, in_b, out_w, out_b):
    n, d_in = x.shape
    e = edge_index.shape[1]
    d_out = W3.shape[1]

    tile_e = NC * NS * NSB_STEP
    e_pad = _cdiv(e, tile_e) * tile_e
    n_pad = _cdiv(n, NS * 8) * NS * 8
    if n_pad == n:
        n_pad += NS * 8
    rpt = n_pad // NS
    assert rpt % 8 == 0

    src = edge_index[0].astype(jnp.int32)
    dst = edge_index[1].astype(jnp.int32)
    pad_i = jnp.arange(e_pad - e, dtype=jnp.int32)
    src_p = jnp.concatenate([src, pad_i % n]).reshape(e_pad // CH, CH)
    dst_p = jnp.concatenate([dst, n + pad_i % (n_pad - n)]).reshape(
        e_pad // CH, CH)

    ones_r = jnp.ones((CH,), jnp.float32)
    zer1 = jnp.zeros((rpt,), jnp.float32)
    zer_w = jnp.zeros((2 * CH, d_in), jnp.float32)

    deg0, deg1 = _make_deg(n_pad, e_pad)(dst_p, ones_r, zer1)
    deg0 = deg0[:n].reshape(n, 1)
    deg1 = deg1[:n].reshape(n, 1)

    dinv, h1p = _tc_mm_scale(x, W1, deg0, deg1)

    agg_w = _make_agg(n, n_pad, d_in, e_pad)

    p0, p1 = agg_w(h1p, src_p, dst_p, zer_w)
    h2p = _tc_mid(p0, p1, dinv, b1.reshape(1, -1), g1.reshape(1, -1),
                  be1.reshape(1, -1), W2, din=d_in)

    p0, p1 = agg_w(h2p, src_p, dst_p, zer_w)
    h3t = _tc_mid(p0, p1, dinv, b2.reshape(1, -1), g2.reshape(1, -1),
                  be2.reshape(1, -1), din=d_in)

    p0, p1 = agg_w(h3t, src_p, dst_p, zer_w)

    wv_t = in_w[2 * d_out:3 * d_out, :].T
    bv = in_b[2 * d_out:3 * d_out].reshape(1, -1)
    out = _tc_final(p0, p1, dinv, W3, b3.reshape(1, -1), wv_t, bv,
                    out_w.T, out_b.reshape(1, -1))
    return out


# speedup vs baseline: 27.0081x; 1.0272x over previous
import functools

import jax
import jax.numpy as jnp
from jax import lax
from jax.experimental import pallas as pl
from jax.experimental.pallas import tpu as pltpu
from jax.experimental.pallas import tpu_sc as plsc

NC = 2
NS = 16
CH = 128
KW = 2
WE = CH * KW
SBW = 8
NSB_STEP = 2 * SBW * CH


def _cdiv(a, b):
    return (a + b - 1) // b


def _make_deg(n_pad, e_pad):
    ept = e_pad // (NC * NS)
    n_win = ept // WE
    rpt = n_pad // NS
    mesh = plsc.VectorSubcoreMesh(core_axis_name="c", subcore_axis_name="s")

    nsb = (ept // CH) // SBW
    assert nsb % 2 == 0

    @functools.partial(
        pl.kernel,
        out_type=(jax.ShapeDtypeStruct((n_pad,), jnp.float32),
                  jax.ShapeDtypeStruct((n_pad,), jnp.float32)),
        mesh=mesh,
        scratch_types=[
            pltpu.VMEM_SHARED((n_pad,), jnp.float32),
            pltpu.VMEM((2, SBW, CH), jnp.int32),
            pltpu.VMEM((CH,), jnp.float32),
            pltpu.VMEM((rpt,), jnp.float32),
            pltpu.SemaphoreType.DMA,
            pltpu.SemaphoreType.DMA,
        ],
    )
    def deg(dstr, onesr, zerr, out0, out1, acc, dstb, onesb, stage,
            isem, ssem):
        c = lax.axis_index("c")
        s = lax.axis_index("s")
        pltpu.sync_copy(onesr, onesb)
        pltpu.sync_copy(zerr, stage)
        pltpu.sync_copy(stage, acc.at[pl.ds(s * rpt, rpt)])
        plsc.subcore_barrier()

        rbase = (c * NS + s) * (ept // CH)
        pltpu.sync_copy(dstr.at[pl.ds(rbase, SBW)], dstb.at[0])

        def outer(o, carry):
            for p in (0, 1):
                sb = 2 * o + p
                noff = rbase + lax.rem(sb + 1, nsb) * SBW
                i0 = pltpu.async_copy(dstr.at[pl.ds(noff, SBW)],
                                      dstb.at[1 - p], isem)
                sd = [pltpu.async_copy(onesb, acc.at[dstb.at[p, k]], ssem,
                                       add=True)
                      for k in range(SBW)]
                for d_ in sd:
                    d_.wait()
                i0.wait()
            return carry

        lax.fori_loop(0, nsb // 2, outer, 0)
        plsc.subcore_barrier()
        pltpu.sync_copy(acc.at[pl.ds(s * rpt, rpt)], stage)

        @pl.when(c == 0)
        def _():
            pltpu.sync_copy(stage, out0.at[pl.ds(s * rpt, rpt)])

        @pl.when(c == 1)
        def _():
            pltpu.sync_copy(stage, out1.at[pl.ds(s * rpt, rpt)])

    return deg


def _make_agg(n, n_pad, d, e_pad):
    ept = e_pad // (NC * NS)
    n_win = ept // CH
    nsb = n_win // SBW
    assert nsb % 2 == 0 and nsb * SBW == n_win
    rpt = n_pad // NS
    stg = 2 * CH
    chunks = [(off, min(stg, rpt - off)) for off in range(0, rpt, stg)]
    mesh = plsc.VectorSubcoreMesh(core_axis_name="c", subcore_axis_name="s")

    @functools.partial(
        pl.kernel,
        out_type=(jax.ShapeDtypeStruct((n_pad, d), jnp.float32),
                  jax.ShapeDtypeStruct((n_pad, d), jnp.float32)),
        mesh=mesh,
        scratch_types=[
            pltpu.VMEM_SHARED((n_pad, d), jnp.float32),
            pltpu.VMEM((2, SBW, CH), jnp.int32),
            pltpu.VMEM((2, SBW, CH), jnp.int32),
            pltpu.VMEM((stg, d), jnp.float32),
            pltpu.SemaphoreType.DMA,
            pltpu.SemaphoreType.DMA,
            pltpu.SemaphoreType.DMA,
        ],
    )
    def agg(table, srcr, dstr, zerr, out0, out1, acc, srcb, dstb, rows,
            isem, gsem, ssem):
        c = lax.axis_index("c")
        s = lax.axis_index("s")
        last_full = n // rpt
        rem_rows = n - last_full * rpt
        pad_rows = n_pad - n
        assert rem_rows % 8 == 0 and pad_rows % 8 == 0 and pad_rows <= 2 * CH

        rbase = (c * NS + s) * n_win
        p0d = pltpu.async_copy(srcr.at[pl.ds(rbase, SBW)], srcb.at[0], isem)
        p1d = pltpu.async_copy(dstr.at[pl.ds(rbase, SBW)], dstb.at[0], isem)

        @pl.when(c == 0)
        def _():
            @pl.when(s < last_full)
            def _():
                pltpu.async_copy(table.at[pl.ds(s * rpt, rpt)],
                                 acc.at[pl.ds(s * rpt, rpt)], gsem).wait()

            @pl.when(s == last_full)
            def _():
                if rem_rows:
                    d0 = pltpu.async_copy(
                        table.at[pl.ds(last_full * rpt, rem_rows)],
                        acc.at[pl.ds(last_full * rpt, rem_rows)], gsem)
                d1 = pltpu.async_copy(zerr.at[pl.ds(0, pad_rows)],
                                      acc.at[pl.ds(n, pad_rows)], gsem)
                if rem_rows:
                    d0.wait()
                d1.wait()

        @pl.when(c == 1)
        def _():
            ds_ = [pltpu.async_copy(zerr.at[pl.ds(0, sz)],
                                    acc.at[pl.ds(s * rpt + off, sz)], gsem)
                   for off, sz in chunks]
            for d_ in ds_:
                d_.wait()
        p0d.wait()
        p1d.wait()
        plsc.subcore_barrier()

        def outer(o, carry):
            for p in (0, 1):
                sb = 2 * o + p
                nxt = lax.rem(sb + 1, nsb)
                noff = rbase + nxt * SBW
                i0 = pltpu.async_copy(srcr.at[pl.ds(noff, SBW)],
                                      srcb.at[1 - p], isem)
                i1 = pltpu.async_copy(dstr.at[pl.ds(noff, SBW)],
                                      dstb.at[1 - p], isem)
                gd = [None] * SBW
                sd = [None] * SBW
                for k in range(SBW):
                    b = k % 2
                    if k >= 2:
                        sd[k - 2].wait()
                    gd[k] = pltpu.async_copy(
                        table.at[srcb.at[p, k]],
                        rows.at[pl.ds(b * CH, CH)], gsem)
                    if k >= 1:
                        gd[k - 1].wait()
                        sd[k - 1] = pltpu.async_copy(
                            rows.at[pl.ds(((k - 1) % 2) * CH, CH)],
                            acc.at[dstb.at[p, k - 1]], ssem, add=True)
                gd[SBW - 1].wait()
                sd[SBW - 1] = pltpu.async_copy(
                    rows.at[pl.ds(((SBW - 1) % 2) * CH, CH)],
                    acc.at[dstb.at[p, SBW - 1]], ssem, add=True)
                sd[SBW - 2].wait()
                sd[SBW - 1].wait()
                i0.wait()
                i1.wait()
            return carry

        lax.fori_loop(0, nsb // 2, outer, 0)
        plsc.subcore_barrier()

        @pl.when(c == 0)
        def _():
            pltpu.sync_copy(acc.at[pl.ds(s * rpt, rpt)],
                            out0.at[pl.ds(s * rpt, rpt)])

        @pl.when(c == 1)
        def _():
            pltpu.sync_copy(acc.at[pl.ds(s * rpt, rpt)],
                            out1.at[pl.ds(s * rpt, rpt)])

    return agg


_BR = 2000


def _mm_kernel(x_ref, w_ref, d0_ref, d1_ref, dinv_ref, hp_ref):
    dinv = 1.0 / jnp.sqrt(d0_ref[...] + d1_ref[...] + 1.0)
    dinv_ref[...] = dinv
    hp_ref[...] = jnp.dot(x_ref[...], w_ref[...],
                          preferred_element_type=jnp.float32) * dinv


def _tc_mm_scale(x, w, deg0, deg1):
    n, din = x.shape
    dout = w.shape[1]
    grid = (n // _BR,)
    return pl.pallas_call(
        _mm_kernel,
        grid=grid,
        in_specs=[pl.BlockSpec((_BR, din), lambda i: (i, 0)),
                  pl.BlockSpec((din, dout), lambda i: (0, 0)),
                  pl.BlockSpec((_BR, 1), lambda i: (i, 0)),
                  pl.BlockSpec((_BR, 1), lambda i: (i, 0))],
        out_specs=[pl.BlockSpec((_BR, 1), lambda i: (i, 0)),
                   pl.BlockSpec((_BR, dout), lambda i: (i, 0))],
        out_shape=[jax.ShapeDtypeStruct((n, 1), jnp.float32),
                   jax.ShapeDtypeStruct((n, dout), jnp.float32)],
    )(x, w, deg0, deg1)


_BN_SCALE = 1.0 / (1.0 + 1e-5) ** 0.5


def _mid_kernel(with_mm, *refs):
    if with_mm:
        p0_ref, p1_ref, dinv_ref, b_ref, g_ref, be_ref, w_ref, o_ref = refs
    else:
        p0_ref, p1_ref, dinv_ref, b_ref, g_ref, be_ref, o_ref = refs
    dinv = dinv_ref[...]
    t = (p0_ref[...] + p1_ref[...]) * dinv + b_ref[...]
    u = jnp.maximum(g_ref[...] * t * _BN_SCALE + be_ref[...], 0.0)
    if with_mm:
        u = jnp.dot(u, w_ref[...], preferred_element_type=jnp.float32)
    o_ref[...] = u * dinv


def _tc_mid(p0, p1, dinv, b, g, be, w=None, din=None):
    n = dinv.shape[0]
    dout = w.shape[1] if w is not None else din
    grid = (n // _BR,)
    in_specs = [pl.BlockSpec((_BR, din), lambda i: (i, 0)),
                pl.BlockSpec((_BR, din), lambda i: (i, 0)),
                pl.BlockSpec((_BR, 1), lambda i: (i, 0)),
                pl.BlockSpec((1, din), lambda i: (0, 0)),
                pl.BlockSpec((1, din), lambda i: (0, 0)),
                pl.BlockSpec((1, din), lambda i: (0, 0))]
    args = [p0, p1, dinv, b, g, be]
    if w is not None:
        in_specs.append(pl.BlockSpec((din, dout), lambda i: (0, 0)))
        args.append(w)
    return pl.pallas_call(
        functools.partial(_mid_kernel, w is not None),
        grid=grid,
        in_specs=in_specs,
        out_specs=pl.BlockSpec((_BR, dout), lambda i: (i, 0)),
        out_shape=jax.ShapeDtypeStruct((n, dout), jnp.float32),
    )(*args)


def _final_kernel(p0_ref, p1_ref, dinv_ref, w3_ref, b_ref, wv_ref,
                  bv_ref, ow_ref, ob_ref, o_ref):
    t = (p0_ref[...] + p1_ref[...]) * dinv_ref[...]
    h = jnp.dot(t, w3_ref[...], preferred_element_type=jnp.float32) \
        + b_ref[...]
    v = jnp.dot(h, wv_ref[...], preferred_element_type=jnp.float32) \
        + bv_ref[...]
    o_ref[...] = h + jnp.dot(v, ow_ref[...],
                             preferred_element_type=jnp.float32) + ob_ref[...]


def _tc_final(p0, p1, dinv, w3, b, wv_t, bv, ow_t, ob):
    n = dinv.shape[0]
    din, d = w3.shape
    grid = (n // _BR,)
    return pl.pallas_call(
        _final_kernel,
        grid=grid,
        in_specs=[pl.BlockSpec((_BR, din), lambda i: (i, 0)),
                  pl.BlockSpec((_BR, din), lambda i: (i, 0)),
                  pl.BlockSpec((_BR, 1), lambda i: (i, 0)),
                  pl.BlockSpec((din, d), lambda i: (0, 0)),
                  pl.BlockSpec((1, d), lambda i: (0, 0)),
                  pl.BlockSpec((d, d), lambda i: (0, 0)),
                  pl.BlockSpec((1, d), lambda i: (0, 0)),
                  pl.BlockSpec((d, d), lambda i: (0, 0)),
                  pl.BlockSpec((1, d), lambda i: (0, 0))],
        out_specs=pl.BlockSpec((_BR, d), lambda i: (i, 0)),
        out_shape=jax.ShapeDtypeStruct((n, d), jnp.float32),
    )(p0, p1, dinv, w3, b, wv_t, bv, ow_t, ob)


def kernel(x, edge_index, W1, b1, g1, be1, W2, b2, g2, be2, W3, b3,
           in_w, in_b, out_w, out_b):
    n, d_in = x.shape
    e = edge_index.shape[1]
    d_out = W3.shape[1]

    tile_e = NC * NS * NSB_STEP
    e_pad = _cdiv(e, tile_e) * tile_e
    n_pad = _cdiv(n, NS * 8) * NS * 8
    if n_pad == n:
        n_pad += NS * 8
    rpt = n_pad // NS
    assert rpt % 8 == 0

    src = edge_index[0].astype(jnp.int32)
    dst = edge_index[1].astype(jnp.int32)
    pad_i = jnp.arange(e_pad - e, dtype=jnp.int32)
    src_p = jnp.concatenate([src, pad_i % n]).reshape(e_pad // CH, CH)
    dst_p = jnp.concatenate([dst, n + pad_i % (n_pad - n)]).reshape(
        e_pad // CH, CH)

    ones_r = jnp.ones((CH,), jnp.float32)
    zer1 = jnp.zeros((rpt,), jnp.float32)
    zer_w = jnp.zeros((2 * CH, d_in), jnp.float32)

    deg0, deg1 = _make_deg(n_pad, e_pad)(dst_p, ones_r, zer1)
    deg0 = deg0.reshape(n_pad, 1)
    deg1 = deg1.reshape(n_pad, 1)

    dinv, h1p = _tc_mm_scale(x, W1, deg0, deg1)

    agg_w = _make_agg(n, n_pad, d_in, e_pad)

    p0, p1 = agg_w(h1p, src_p, dst_p, zer_w)
    h2p = _tc_mid(p0, p1, dinv, b1.reshape(1, -1), g1.reshape(1, -1),
                  be1.reshape(1, -1), W2, din=d_in)

    p0, p1 = agg_w(h2p, src_p, dst_p, zer_w)
    h3t = _tc_mid(p0, p1, dinv, b2.reshape(1, -1), g2.reshape(1, -1),
                  be2.reshape(1, -1), din=d_in)

    p0, p1 = agg_w(h3t, src_p, dst_p, zer_w)

    wv_t = in_w[2 * d_out:3 * d_out, :].T
    bv = in_b[2 * d_out:3 * d_out].reshape(1, -1)
    out = _tc_final(p0, p1, dinv, W3, b3.reshape(1, -1), wv_t, bv,
                    out_w.T, out_b.reshape(1, -1))
    return out


# speedup vs baseline: 27.4499x; 1.0164x over previous
import functools

import jax
import jax.numpy as jnp
from jax import lax
from jax.experimental import pallas as pl
from jax.experimental.pallas import tpu as pltpu
from jax.experimental.pallas import tpu_sc as plsc

NC = 2
NS = 16
CH = 128
KW = 2
WE = CH * KW
SBW = 8
NSB_STEP = 2 * SBW * CH


def _cdiv(a, b):
    return (a + b - 1) // b


def _make_deg(n_pad, e_pad):
    ept = e_pad // (NC * NS)
    n_win = ept // WE
    rpt = n_pad // NS
    mesh = plsc.VectorSubcoreMesh(core_axis_name="c", subcore_axis_name="s")

    nsb = (ept // CH) // SBW
    assert nsb % 2 == 0

    @functools.partial(
        pl.kernel,
        out_type=(jax.ShapeDtypeStruct((n_pad,), jnp.float32),
                  jax.ShapeDtypeStruct((n_pad,), jnp.float32)),
        mesh=mesh,
        scratch_types=[
            pltpu.VMEM_SHARED((n_pad,), jnp.float32),
            pltpu.VMEM((2, SBW, CH), jnp.int32),
            pltpu.VMEM((CH,), jnp.float32),
            pltpu.VMEM((rpt,), jnp.float32),
            pltpu.SemaphoreType.DMA,
            pltpu.SemaphoreType.DMA,
        ],
    )
    def deg(dstr, onesr, zerr, out0, out1, acc, dstb, onesb, stage,
            isem, ssem):
        c = lax.axis_index("c")
        s = lax.axis_index("s")
        pltpu.sync_copy(onesr, onesb)
        pltpu.sync_copy(zerr, stage)
        pltpu.sync_copy(stage, acc.at[pl.ds(s * rpt, rpt)])
        plsc.subcore_barrier()

        rbase = (c * NS + s) * (ept // CH)
        pltpu.sync_copy(dstr.at[pl.ds(rbase, SBW)], dstb.at[0])

        def outer(o, carry):
            for p in (0, 1):
                sb = 2 * o + p
                noff = rbase + lax.rem(sb + 1, nsb) * SBW
                i0 = pltpu.async_copy(dstr.at[pl.ds(noff, SBW)],
                                      dstb.at[1 - p], isem)
                sd = [pltpu.async_copy(onesb, acc.at[dstb.at[p, k]], ssem,
                                       add=True)
                      for k in range(SBW)]
                for d_ in sd:
                    d_.wait()
                i0.wait()
            return carry

        lax.fori_loop(0, nsb // 2, outer, 0)
        plsc.subcore_barrier()
        pltpu.sync_copy(acc.at[pl.ds(s * rpt, rpt)], stage)

        @pl.when(c == 0)
        def _():
            pltpu.sync_copy(stage, out0.at[pl.ds(s * rpt, rpt)])

        @pl.when(c == 1)
        def _():
            pltpu.sync_copy(stage, out1.at[pl.ds(s * rpt, rpt)])

    return deg


def _make_agg(n, n_pad, d, e_pad):
    ept = e_pad // (NC * NS)
    n_win = ept // CH
    nsb = n_win // SBW
    assert nsb % 2 == 0 and nsb * SBW == n_win
    rpt = n_pad // NS
    stg = 2 * CH
    chunks = [(off, min(stg, rpt - off)) for off in range(0, rpt, stg)]
    mesh = plsc.VectorSubcoreMesh(core_axis_name="c", subcore_axis_name="s")

    @functools.partial(
        pl.kernel,
        out_type=(jax.ShapeDtypeStruct((n_pad, d), jnp.float32),
                  jax.ShapeDtypeStruct((n_pad, d), jnp.float32)),
        mesh=mesh,
        scratch_types=[
            pltpu.VMEM_SHARED((n_pad, d), jnp.float32),
            pltpu.VMEM((2, SBW, CH), jnp.int32),
            pltpu.VMEM((2, SBW, CH), jnp.int32),
            pltpu.VMEM((stg, d), jnp.float32),
            pltpu.SemaphoreType.DMA,
            pltpu.SemaphoreType.DMA,
            pltpu.SemaphoreType.DMA,
        ],
    )
    def agg(table, srcr, dstr, zerr, out0, out1, acc, srcb, dstb, rows,
            isem, gsem, ssem):
        c = lax.axis_index("c")
        s = lax.axis_index("s")
        last_full = n // rpt
        rem_rows = n - last_full * rpt
        pad_rows = n_pad - n
        assert rem_rows % 8 == 0 and pad_rows % 8 == 0 and pad_rows <= 2 * CH

        rbase = (c * NS + s) * n_win
        p0d = pltpu.async_copy(srcr.at[pl.ds(rbase, SBW)], srcb.at[0], isem)
        p1d = pltpu.async_copy(dstr.at[pl.ds(rbase, SBW)], dstb.at[0], isem)

        @pl.when(c == 0)
        def _():
            @pl.when(s < last_full)
            def _():
                pltpu.async_copy(table.at[pl.ds(s * rpt, rpt)],
                                 acc.at[pl.ds(s * rpt, rpt)], gsem).wait()

            @pl.when(s == last_full)
            def _():
                if rem_rows:
                    d0 = pltpu.async_copy(
                        table.at[pl.ds(last_full * rpt, rem_rows)],
                        acc.at[pl.ds(last_full * rpt, rem_rows)], gsem)
                d1 = pltpu.async_copy(zerr.at[pl.ds(0, pad_rows)],
                                      acc.at[pl.ds(n, pad_rows)], gsem)
                if rem_rows:
                    d0.wait()
                d1.wait()

        @pl.when(c == 1)
        def _():
            ds_ = [pltpu.async_copy(zerr.at[pl.ds(0, sz)],
                                    acc.at[pl.ds(s * rpt + off, sz)], gsem)
                   for off, sz in chunks]
            for d_ in ds_:
                d_.wait()
        p0d.wait()
        p1d.wait()
        plsc.subcore_barrier()

        def outer(o, carry):
            for p in (0, 1):
                sb = 2 * o + p
                nxt = lax.rem(sb + 1, nsb)
                noff = rbase + nxt * SBW
                i0 = pltpu.async_copy(srcr.at[pl.ds(noff, SBW)],
                                      srcb.at[1 - p], isem)
                i1 = pltpu.async_copy(dstr.at[pl.ds(noff, SBW)],
                                      dstb.at[1 - p], isem)
                gd = [None] * SBW
                sd = [None] * SBW
                for k in range(SBW):
                    b = k % 2
                    if k >= 2:
                        sd[k - 2].wait()
                    gd[k] = pltpu.async_copy(
                        table.at[srcb.at[p, k]],
                        rows.at[pl.ds(b * CH, CH)], gsem)
                    if k >= 1:
                        gd[k - 1].wait()
                        sd[k - 1] = pltpu.async_copy(
                            rows.at[pl.ds(((k - 1) % 2) * CH, CH)],
                            acc.at[dstb.at[p, k - 1]], ssem, add=True)
                gd[SBW - 1].wait()
                sd[SBW - 1] = pltpu.async_copy(
                    rows.at[pl.ds(((SBW - 1) % 2) * CH, CH)],
                    acc.at[dstb.at[p, SBW - 1]], ssem, add=True)
                sd[SBW - 2].wait()
                sd[SBW - 1].wait()
                i0.wait()
                i1.wait()
            return carry

        lax.fori_loop(0, nsb // 2, outer, 0)
        plsc.subcore_barrier()

        @pl.when(c == 0)
        def _():
            pltpu.sync_copy(acc.at[pl.ds(s * rpt, rpt)],
                            out0.at[pl.ds(s * rpt, rpt)])

        @pl.when(c == 1)
        def _():
            pltpu.sync_copy(acc.at[pl.ds(s * rpt, rpt)],
                            out1.at[pl.ds(s * rpt, rpt)])

    return agg


_BR = 2000


def _mm_kernel(x_ref, w_ref, d0_ref, d1_ref, dinv_ref, hp_ref):
    dinv = 1.0 / jnp.sqrt(d0_ref[...] + d1_ref[...] + 1.0)
    dinv_ref[...] = dinv
    hp_ref[...] = jnp.dot(x_ref[...], w_ref[...],
                          preferred_element_type=jnp.float32) * dinv


def _tc_mm_scale(x, w, deg0, deg1):
    n, din = x.shape
    dout = w.shape[1]
    grid = (n // _BR,)
    return pl.pallas_call(
        _mm_kernel,
        grid=grid,
        in_specs=[pl.BlockSpec((_BR, din), lambda i: (i, 0)),
                  pl.BlockSpec((din, dout), lambda i: (0, 0)),
                  pl.BlockSpec((_BR, 1), lambda i: (i, 0)),
                  pl.BlockSpec((_BR, 1), lambda i: (i, 0))],
        out_specs=[pl.BlockSpec((_BR, 1), lambda i: (i, 0)),
                   pl.BlockSpec((_BR, dout), lambda i: (i, 0))],
        out_shape=[jax.ShapeDtypeStruct((n, 1), jnp.float32),
                   jax.ShapeDtypeStruct((n, dout), jnp.float32)],
    )(x, w, deg0, deg1)


_BN_SCALE = 1.0 / (1.0 + 1e-5) ** 0.5


def _mid_kernel(with_mm, *refs):
    if with_mm:
        p0_ref, p1_ref, dinv_ref, b_ref, g_ref, be_ref, w_ref, o_ref = refs
    else:
        p0_ref, p1_ref, dinv_ref, b_ref, g_ref, be_ref, o_ref = refs
    dinv = dinv_ref[...]
    t = (p0_ref[...] + p1_ref[...]) * dinv + b_ref[...]
    u = jnp.maximum(g_ref[...] * t * _BN_SCALE + be_ref[...], 0.0)
    if with_mm:
        u = jnp.dot(u, w_ref[...], preferred_element_type=jnp.float32)
    o_ref[...] = u * dinv


def _tc_mid(p0, p1, dinv, b, g, be, w=None, din=None):
    n = dinv.shape[0]
    dout = w.shape[1] if w is not None else din
    grid = (n // _BR,)
    in_specs = [pl.BlockSpec((_BR, din), lambda i: (i, 0)),
                pl.BlockSpec((_BR, din), lambda i: (i, 0)),
                pl.BlockSpec((_BR, 1), lambda i: (i, 0)),
                pl.BlockSpec((1, din), lambda i: (0, 0)),
                pl.BlockSpec((1, din), lambda i: (0, 0)),
                pl.BlockSpec((1, din), lambda i: (0, 0))]
    args = [p0, p1, dinv, b, g, be]
    if w is not None:
        in_specs.append(pl.BlockSpec((din, dout), lambda i: (0, 0)))
        args.append(w)
    return pl.pallas_call(
        functools.partial(_mid_kernel, w is not None),
        grid=grid,
        in_specs=in_specs,
        out_specs=pl.BlockSpec((_BR, dout), lambda i: (i, 0)),
        out_shape=jax.ShapeDtypeStruct((n, dout), jnp.float32),
    )(*args)


def _dg_t(w, u):
    return lax.dot_general(w, u, (((0,), (1,)), ((), ())),
                           preferred_element_type=jnp.float32)


def _final_kernel(p0_ref, p1_ref, dinv_ref, w3_ref, b_ref, wv_ref,
                  bv_ref, ow_ref, ob_ref, o_ref):
    u = (p0_ref[...] + p1_ref[...]) * dinv_ref[...]
    h_t = _dg_t(w3_ref[...], u) + b_ref[...]
    v_t = jnp.dot(wv_ref[...], h_t,
                  preferred_element_type=jnp.float32) + bv_ref[...]
    o_ref[...] = h_t + jnp.dot(ow_ref[...], v_t,
                               preferred_element_type=jnp.float32) \
        + ob_ref[...]


def _tc_final(p0, p1, dinv, w3, b, wv, bv, ow, ob):
    n = dinv.shape[0]
    din, d = w3.shape
    return pl.pallas_call(
        _final_kernel,
        grid=(1,),
        in_specs=[pl.BlockSpec((n, din), lambda i: (0, 0)),
                  pl.BlockSpec((n, din), lambda i: (0, 0)),
                  pl.BlockSpec((n, 1), lambda i: (0, 0)),
                  pl.BlockSpec((din, d), lambda i: (0, 0)),
                  pl.BlockSpec((d, 1), lambda i: (0, 0)),
                  pl.BlockSpec((d, d), lambda i: (0, 0)),
                  pl.BlockSpec((d, 1), lambda i: (0, 0)),
                  pl.BlockSpec((d, d), lambda i: (0, 0)),
                  pl.BlockSpec((d, 1), lambda i: (0, 0))],
        out_specs=pl.BlockSpec((d, n), lambda i: (0, 0)),
        out_shape=jax.ShapeDtypeStruct((d, n), jnp.float32),
    )(p0, p1, dinv, w3, b, wv, bv, ow, ob)


def kernel(x, edge_index, W1, b1, g1, be1, W2, b2, g2, be2, W3, b3,
           in_w, in_b, out_w, out_b):
    n, d_in = x.shape
    e = edge_index.shape[1]
    d_out = W3.shape[1]

    tile_e = NC * NS * NSB_STEP
    e_pad = _cdiv(e, tile_e) * tile_e
    n_pad = _cdiv(n, NS * 8) * NS * 8
    if n_pad == n:
        n_pad += NS * 8
    rpt = n_pad // NS
    assert rpt % 8 == 0

    src = edge_index[0].astype(jnp.int32)
    dst = edge_index[1].astype(jnp.int32)
    pad_i = jnp.arange(e_pad - e, dtype=jnp.int32)
    src_p = jnp.concatenate([src, pad_i % n]).reshape(e_pad // CH, CH)
    dst_p = jnp.concatenate([dst, n + pad_i % (n_pad - n)]).reshape(
        e_pad // CH, CH)

    ones_r = jnp.ones((CH,), jnp.float32)
    zer1 = jnp.zeros((rpt,), jnp.float32)
    zer_w = jnp.zeros((2 * CH, d_in), jnp.float32)

    deg0, deg1 = _make_deg(n_pad, e_pad)(dst_p, ones_r, zer1)
    deg0 = deg0.reshape(n_pad, 1)
    deg1 = deg1.reshape(n_pad, 1)

    dinv, h1p = _tc_mm_scale(x, W1, deg0, deg1)

    agg_w = _make_agg(n, n_pad, d_in, e_pad)

    p0, p1 = agg_w(h1p, src_p, dst_p, zer_w)
    h2p = _tc_mid(p0, p1, dinv, b1.reshape(1, -1), g1.reshape(1, -1),
                  be1.reshape(1, -1), W2, din=d_in)

    p0, p1 = agg_w(h2p, src_p, dst_p, zer_w)
    h3t = _tc_mid(p0, p1, dinv, b2.reshape(1, -1), g2.reshape(1, -1),
                  be2.reshape(1, -1), din=d_in)

    p0, p1 = agg_w(h3t, src_p, dst_p, zer_w)

    wv = in_w[2 * d_out:3 * d_out, :]
    bv = in_b[2 * d_out:3 * d_out].reshape(-1, 1)
    out_t = _tc_final(p0, p1, dinv, W3, b3.reshape(-1, 1), wv, bv,
                      out_w, out_b.reshape(-1, 1))
    return out_t.T
